# baseline (speedup 1.0000x reference)
import functools

import jax
import jax.numpy as jnp
from jax import lax
from jax.experimental import pallas as pl
from jax.experimental.pallas import tpu as pltpu
from jax.experimental.pallas import tpu_sc as plsc

NC, NS = 2, 16
NW = NC * NS
IDXW = 128


def _sc_mesh():
  return plsc.VectorSubcoreMesh(
      core_axis_name="c", subcore_axis_name="s",
      num_cores=NC, num_subcores=NS)


def _sc_gather(table, idx3, steps):
  _, D = table.shape
  N = NW * steps * IDXW

  @functools.partial(
      pl.kernel,
      out_type=jax.ShapeDtypeStruct((N, D), jnp.float32),
      mesh=_sc_mesh(),
      compiler_params=pltpu.CompilerParams(use_tc_tiling_on_sc=False),
      scratch_types=[
          pltpu.VMEM((steps, IDXW), jnp.int32),
          pltpu.VMEM((IDXW, D), jnp.float32),
          pltpu.SemaphoreType.DMA,
      ],
  )
  def k(table_hbm, idx_hbm, out_hbm, idx_v, rows_v, sem):
    w = lax.axis_index("s") * NC + lax.axis_index("c")
    pltpu.sync_copy(idx_hbm.at[w], idx_v)

    def body(s, carry):
      pltpu.async_copy(table_hbm.at[idx_v.at[s]], rows_v, sem).wait()
      pltpu.sync_copy(rows_v, out_hbm.at[pl.ds((w * steps + s) * IDXW, IDXW)])
      return carry

    lax.fori_loop(0, steps, body, 0)

  return k(table, idx3)


def _sc_scatter(vals2, mi, di, num_m, num_d):
  BP = vals2.shape[0] * IDXW
  per_tile = BP // NS
  CH = 1024
  KD = CH // IDXW
  nch = per_tile // CH
  rows_half = num_m // 2
  flat_n = (num_m + 1) * num_d
  trash = num_m * num_d
  zspan = rows_half * num_d // NS
  ZCH = 25000
  zn = zspan // ZCH
  assert zn * ZCH == zspan and nch * CH == per_tile

  @functools.partial(
      pl.kernel,
      out_type=jax.ShapeDtypeStruct((flat_n,), jnp.float32),
      mesh=_sc_mesh(),
      scratch_types=[
          pltpu.VMEM((ZCH + 24, ), jnp.float32),
          pltpu.VMEM((CH,), jnp.int32),
          pltpu.VMEM((CH,), jnp.int32),
          pltpu.VMEM((KD, IDXW), jnp.float32),
          pltpu.VMEM((KD, IDXW), jnp.int32),
          pltpu.SemaphoreType.DMA,
      ],
  )
  def k(val_hbm, mi_hbm, di_hbm, out_hbm, zb, mi_v, di_v, val_v, idx_v, sem):
    c = lax.axis_index("c")
    t = lax.axis_index("s")

    def zfill(i, carry):
      zb[pl.ds(i * 16, 16)] = jnp.zeros((16,), jnp.float32)
      return carry

    lax.fori_loop(0, (ZCH + 24) // 16, zfill, 0)
    zbase = c * (rows_half * num_d) + t * zspan

    def zdma(i, carry):
      pltpu.sync_copy(zb.at[pl.ds(0, ZCH)],
                      out_hbm.at[pl.ds(zbase + i * ZCH, ZCH)])
      return carry

    lax.fori_loop(0, zn, zdma, 0)
    plsc.subcore_barrier()

    row_lo = c * rows_half
    row_hi = row_lo + rows_half

    def chunk(ci, carry):
      off = t * per_tile + ci * CH
      pltpu.sync_copy(mi_hbm.at[pl.ds(off, CH)], mi_v)
      pltpu.sync_copy(di_hbm.at[pl.ds(off, CH)], di_v)
      pltpu.sync_copy(val_hbm.at[pl.ds(t * (per_tile // IDXW) + ci * KD, KD)],
                      val_v)

      def vec(j, carry2):
        def vec16(i, carry3):
          m = mi_v[pl.ds(j * IDXW + i * 16, 16)]
          d = di_v[pl.ds(j * IDXW + i * 16, 16)]
          inreg = (m >= row_lo) & (m < row_hi)
          idx_v[j, pl.ds(i * 16, 16)] = jnp.where(
              inreg, m * num_d + d, trash + d)
          return carry3
        lax.fori_loop(0, IDXW // 16, vec16, 0)
        return carry2

      lax.fori_loop(0, KD, vec, 0)
      copies = [
          pltpu.async_copy(val_v.at[j], out_hbm.at[idx_v.at[j]], sem)
          for j in range(KD)
      ]
      for cp in copies:
        cp.wait()
      return carry

    lax.fori_loop(0, nch, chunk, 0)

  return k(vals2, mi, di)


def _tc_mlp(me, de, lab, W_m, b_m, W_d, b_d, W1m, b1m, W2m, b2m,
            W1d, b1d, W2d, b2d, b_real):
  BP = me.shape[0]
  BLK = 4096
  G = BP // BLK
  h = W_m.shape[0]
  s_dim = W2m.shape[0]

  def body(me_ref, de_ref, lab_ref, Wm_r, bm_r, Wd_r, bd_r, W1m_r, b1m_r,
           W2m_r, b2m_r, W1d_r, b1d_r, W2d_r, b2d_r,
           pay_ref, msds_ref, bce_ref):
    i = pl.program_id(0)

    def dn(x, w_ref):
      return lax.dot_general(x, w_ref[...], (((1,), (1,)), ((), ())),
                             preferred_element_type=jnp.float32,
                             precision=lax.Precision.HIGHEST)

    pm = dn(me_ref[...], Wm_r) + bm_r[...]
    pd = dn(de_ref[...], Wd_r) + bd_r[...]
    ms = dn(jnp.maximum(dn(pm, W1m_r) + b1m_r[...], 0.0), W2m_r) + b2m_r[...]
    ds = dn(jnp.maximum(dn(pd, W1d_r) + b1d_r[...], 0.0), W2d_r) + b2d_r[...]
    num = jnp.sum(ms * ds, axis=1, keepdims=True)
    den = jnp.sqrt(jnp.sum(ms * ms, axis=1, keepdims=True)) * \
        jnp.sqrt(jnp.sum(ds * ds, axis=1, keepdims=True))
    x = num / den
    pay_ref[...] = x
    msds_ref[...] = jnp.concatenate([ms, ds], axis=1)
    y = lab_ref[...]
    bce = jnp.maximum(x, 0.0) - x * y + jnp.log1p(jnp.exp(-jnp.abs(x)))
    ridx = i * BLK + lax.broadcasted_iota(jnp.int32, (BLK, 1), 0)
    s = jnp.sum(jnp.where(ridx < b_real, bce, 0.0), axis=(0, 1),
                keepdims=True)
    bce_ref[...] = jnp.where(i == 0, s, bce_ref[...] + s)

  rep = lambda shape: pl.BlockSpec(shape, lambda i: (0, 0))
  return pl.pallas_call(
      body,
      grid=(G,),
      in_specs=[
          pl.BlockSpec((BLK, me.shape[1]), lambda i: (i, 0)),
          pl.BlockSpec((BLK, de.shape[1]), lambda i: (i, 0)),
          pl.BlockSpec((BLK, 1), lambda i: (i, 0)),
          rep(W_m.shape), rep((1, h)), rep(W_d.shape), rep((1, h)),
          rep(W1m.shape), rep((1, h)), rep(W2m.shape), rep((1, s_dim)),
          rep(W1d.shape), rep((1, h)), rep(W2d.shape), rep((1, s_dim)),
      ],
      out_specs=[
          pl.BlockSpec((BLK, 1), lambda i: (i, 0)),
          pl.BlockSpec((BLK, 2 * s_dim), lambda i: (i, 0)),
          pl.BlockSpec((1, 1), lambda i: (0, 0)),
      ],
      out_shape=[
          jax.ShapeDtypeStruct((BP, 1), jnp.float32),
          jax.ShapeDtypeStruct((BP, 2 * s_dim), jnp.float32),
          jax.ShapeDtypeStruct((1, 1), jnp.float32),
      ],
  )(me, de, lab, W_m, b_m.reshape(1, h), W_d, b_d.reshape(1, h),
    W1m, b1m.reshape(1, h), W2m, b2m.reshape(1, s_dim),
    W1d, b1d.reshape(1, h), W2d, b2d.reshape(1, s_dim))


def _tc_argmax(mat, num_m, num_d):
  RB = 1000
  G = num_m // RB

  def body(m_ref, bi_ref):
    v = m_ref[...]
    mx = jnp.max(v, axis=1, keepdims=True)
    col = lax.broadcasted_iota(jnp.int32, v.shape, 1)
    bi_ref[...] = jnp.min(jnp.where(v == mx, col, num_d), axis=1,
                          keepdims=True)

  return pl.pallas_call(
      body,
      grid=(G,),
      in_specs=[pl.BlockSpec((RB, num_d), lambda i: (i, 0))],
      out_specs=pl.BlockSpec((RB, 1), lambda i: (i, 0)),
      out_shape=jax.ShapeDtypeStruct((num_m, 1), jnp.int32),
  )(mat)


def _tc_final(msds, big, bce, b_real, s_dim):
  BP = msds.shape[0]
  BLK = 4096
  G = BP // BLK

  def body(a_ref, g_ref, bce_ref, out_ref):
    i = pl.program_id(0)
    d = a_ref[...] - g_ref[...]
    ridx = i * BLK + lax.broadcasted_iota(jnp.int32, (BLK, 1), 0)
    s = jnp.sum(jnp.where(ridx < b_real, d * d, 0.0), axis=(0, 1),
                keepdims=True)
    acc = jnp.where(i == 0, s, out_ref[...] + s)
    out_ref[...] = jnp.where(
        i == G - 1,
        acc / (2.0 * b_real * s_dim) + bce_ref[...] / b_real,
        acc)

  return pl.pallas_call(
      body,
      grid=(G,),
      in_specs=[
          pl.BlockSpec((BLK, 2 * s_dim), lambda i: (i, 0)),
          pl.BlockSpec((BLK, 2 * s_dim), lambda i: (i, 0)),
          pl.BlockSpec((1, 1), lambda i: (0, 0)),
      ],
      out_specs=pl.BlockSpec((1, 1), lambda i: (0, 0)),
      out_shape=jax.ShapeDtypeStruct((1, 1), jnp.float32),
  )(msds, big, bce)


def kernel(miRNA_embeddings, disease_embeddings, miRNA_index, disease_index,
           true_labels, W_m, b_m, W_d, b_d, W_ms1, b_ms1, W_ms2, b_ms2,
           W_ds1, b_ds1, W_ds2, b_ds2):
  num_m = miRNA_embeddings.shape[0]
  num_d = disease_embeddings.shape[0]
  B = miRNA_index.shape[0]
  s_dim = W_ms2.shape[0]

  BP = 16384 * (-(-B // 16384))
  steps = BP // (NW * IDXW)
  pad = BP - B
  mi = miRNA_index.astype(jnp.int32)
  di = disease_index.astype(jnp.int32)
  mi_g = jnp.pad(mi, (0, pad))
  di_g = jnp.pad(di, (0, pad))
  mi_s = jnp.pad(mi, (0, pad), constant_values=num_m)
  lab = jnp.pad(true_labels, (0, pad)).reshape(BP, 1)

  mi2 = mi_g.reshape(NW, steps, IDXW)
  di2 = di_g.reshape(NW, steps, IDXW)

  me = _sc_gather(miRNA_embeddings, mi2, steps)
  de = _sc_gather(disease_embeddings, di2, steps)

  pay, msds, bce = _tc_mlp(me, de, lab, W_m, b_m, W_d, b_d,
                           W_ms1, b_ms1, W_ms2, b_ms2,
                           W_ds1, b_ds1, W_ds2, b_ds2, B)

  matflat = _sc_scatter(pay.reshape(-1, IDXW), mi_s, di_g, num_m, num_d)
  bi = _tc_argmax(matflat.reshape(num_m + 1, num_d), num_m, num_d)

  bi_n = NW * IDXW * (-(-num_m // (NW * IDXW)))
  bi_steps = bi_n // (NW * IDXW)
  bi_pad = jnp.pad(bi.reshape(-1),
                   (0, bi_n - num_m)).reshape(NW, bi_steps, IDXW)
  g2 = _sc_gather(msds, bi_pad, bi_steps)
  big = _sc_gather(g2, mi2, steps)

  loss = _tc_final(msds, big, bce, B, s_dim)
  return (pay.reshape(-1)[:B], loss[0, 0])


# speedup vs baseline: 2.6069x; 2.6069x over previous
import functools

import jax
import jax.numpy as jnp
from jax import lax
from jax.experimental import pallas as pl
from jax.experimental.pallas import tpu as pltpu
from jax.experimental.pallas import tpu_sc as plsc

NC, NS = 2, 16
NW = NC * NS
IDXW = 128


def _sc_mesh():
  return plsc.VectorSubcoreMesh(
      core_axis_name="c", subcore_axis_name="s",
      num_cores=NC, num_subcores=NS)


def _sc_gather(table, idx3, steps):
  _, D = table.shape
  N = NW * steps * IDXW

  @functools.partial(
      pl.kernel,
      out_type=jax.ShapeDtypeStruct((N, D), jnp.float32),
      mesh=_sc_mesh(),
      compiler_params=pltpu.CompilerParams(use_tc_tiling_on_sc=False),
      scratch_types=[
          pltpu.VMEM((steps, IDXW), jnp.int32),
          pltpu.VMEM((IDXW, D), jnp.float32),
          pltpu.SemaphoreType.DMA,
      ],
  )
  def k(table_hbm, idx_hbm, out_hbm, idx_v, rows_v, sem):
    w = lax.axis_index("s") * NC + lax.axis_index("c")
    pltpu.sync_copy(idx_hbm.at[w], idx_v)

    def body(s, carry):
      pltpu.async_copy(table_hbm.at[idx_v.at[s]], rows_v, sem).wait()
      pltpu.sync_copy(rows_v, out_hbm.at[pl.ds((w * steps + s) * IDXW, IDXW)])
      return carry

    lax.fori_loop(0, steps, body, 0)

  return k(table, idx3)


def _sc_segmax(vals3, mi3, di3, steps, T):
  seg = T // NS

  @functools.partial(
      pl.kernel,
      out_type=jax.ShapeDtypeStruct((NC, T), jnp.int32),
      mesh=_sc_mesh(),
      compiler_params=pltpu.CompilerParams(use_tc_tiling_on_sc=False,
                                           needs_layout_passes=False),
      scratch_types=[
          pltpu.VMEM((steps, IDXW), jnp.float32),
          pltpu.VMEM((steps, IDXW), jnp.int32),
          pltpu.VMEM((steps, IDXW), jnp.int32),
          pltpu.VMEM((T,), jnp.int32),
          pltpu.VMEM((seg,), jnp.int32),
          pltpu.VMEM((seg,), jnp.int32),
          pltpu.VMEM_SHARED((NS, T), jnp.int32),
      ],
  )
  def k(val_hbm, mi_hbm, di_hbm, out_hbm, val_v, mi_v, di_v, tbl, mseg,
        tseg, spm):
    c = lax.axis_index("c")
    sid = lax.axis_index("s")
    w = sid * NC + c

    def init(i, carry):
      tbl[pl.ds(i * 16, 16)] = jnp.full((16,), 2047, jnp.int32)
      return carry

    lax.fori_loop(0, T // 16, init, 0)
    pltpu.sync_copy(val_hbm.at[w], val_v)
    pltpu.sync_copy(mi_hbm.at[w], mi_v)
    pltpu.sync_copy(di_hbm.at[w], di_v)

    def row(s, carry):
      def vec(kk, carry2):
        m = mi_v[s, pl.ds(kk * 16, 16)]
        d = di_v[s, pl.ds(kk * 16, 16)]
        v = val_v[s, pl.ds(kk * 16, 16)]
        b = plsc.bitcast(v, jnp.int32)
        key32 = jnp.where(b >= 0, b, b ^ jnp.int32(0x7FFFFFFF))
        key = (key32 & jnp.int32(-2048)) | (jnp.int32(2047) - d)

        def w_cond(mask):
          return jnp.max(jnp.where(mask, 1, 0)) > 0

        def w_body(mask):
          plsc.store_scatter(tbl, [m], key, mask=mask)
          cur = plsc.load_gather(tbl, [m])
          return mask & (cur < key)

        lax.while_loop(w_cond, w_body,
                       key > plsc.load_gather(tbl, [m]))
        return carry2

      lax.fori_loop(0, IDXW // 16, vec, carry)
      return carry

    lax.fori_loop(0, steps, row, 0)

    pltpu.sync_copy(tbl, spm.at[sid])
    plsc.subcore_barrier()
    pltpu.sync_copy(spm.at[0, pl.ds(sid * seg, seg)], mseg)

    def merge(j, carry):
      pltpu.sync_copy(spm.at[j, pl.ds(sid * seg, seg)], tseg)

      def mvec(i, carry2):
        mseg[pl.ds(i * 16, 16)] = jnp.maximum(
            mseg[pl.ds(i * 16, 16)], tseg[pl.ds(i * 16, 16)])
        return carry2

      lax.fori_loop(0, seg // 16, mvec, carry)
      return carry

    lax.fori_loop(1, NS, merge, 0)
    pltpu.sync_copy(mseg, out_hbm.at[c, pl.ds(sid * seg, seg)])

  return k(vals3, mi3, di3)


def _tc_mlp(me, de, lab, W_m, b_m, W_d, b_d, W1m, b1m, W2m, b2m,
            W1d, b1d, W2d, b2d, b_real):
  BP = me.shape[0]
  BLK = 4096
  G = BP // BLK
  h = W_m.shape[0]
  s_dim = W2m.shape[0]

  def body(me_ref, de_ref, lab_ref, Wm_r, bm_r, Wd_r, bd_r, W1m_r, b1m_r,
           W2m_r, b2m_r, W1d_r, b1d_r, W2d_r, b2d_r,
           pay_ref, msds_ref, bce_ref):
    i = pl.program_id(0)

    def dn(x, w_ref):
      return lax.dot_general(x, w_ref[...], (((1,), (1,)), ((), ())),
                             preferred_element_type=jnp.float32,
                             precision=lax.Precision.HIGHEST)

    pm = dn(me_ref[...], Wm_r) + bm_r[...]
    pd = dn(de_ref[...], Wd_r) + bd_r[...]
    ms = dn(jnp.maximum(dn(pm, W1m_r) + b1m_r[...], 0.0), W2m_r) + b2m_r[...]
    ds = dn(jnp.maximum(dn(pd, W1d_r) + b1d_r[...], 0.0), W2d_r) + b2d_r[...]
    num = jnp.sum(ms * ds, axis=1, keepdims=True)
    den = jnp.sqrt(jnp.sum(ms * ms, axis=1, keepdims=True)) * \
        jnp.sqrt(jnp.sum(ds * ds, axis=1, keepdims=True))
    x = num / den
    pay_ref[...] = x
    msds_ref[...] = jnp.concatenate([ms, ds], axis=1)
    y = lab_ref[...]
    bce = jnp.maximum(x, 0.0) - x * y + jnp.log1p(jnp.exp(-jnp.abs(x)))
    ridx = i * BLK + lax.broadcasted_iota(jnp.int32, (BLK, 1), 0)
    s = jnp.sum(jnp.where(ridx < b_real, bce, 0.0), axis=(0, 1),
                keepdims=True)
    bce_ref[...] = jnp.where(i == 0, s, bce_ref[...] + s)

  rep = lambda shape: pl.BlockSpec(shape, lambda i: (0, 0))
  return pl.pallas_call(
      body,
      grid=(G,),
      in_specs=[
          pl.BlockSpec((BLK, me.shape[1]), lambda i: (i, 0)),
          pl.BlockSpec((BLK, de.shape[1]), lambda i: (i, 0)),
          pl.BlockSpec((BLK, 1), lambda i: (i, 0)),
          rep(W_m.shape), rep((1, h)), rep(W_d.shape), rep((1, h)),
          rep(W1m.shape), rep((1, h)), rep(W2m.shape), rep((1, s_dim)),
          rep(W1d.shape), rep((1, h)), rep(W2d.shape), rep((1, s_dim)),
      ],
      out_specs=[
          pl.BlockSpec((BLK, 1), lambda i: (i, 0)),
          pl.BlockSpec((BLK, 2 * s_dim), lambda i: (i, 0)),
          pl.BlockSpec((1, 1), lambda i: (0, 0)),
      ],
      out_shape=[
          jax.ShapeDtypeStruct((BP, 1), jnp.float32),
          jax.ShapeDtypeStruct((BP, 2 * s_dim), jnp.float32),
          jax.ShapeDtypeStruct((1, 1), jnp.float32),
      ],
  )(me, de, lab, W_m, b_m.reshape(1, h), W_d, b_d.reshape(1, h),
    W1m, b1m.reshape(1, h), W2m, b2m.reshape(1, s_dim),
    W1d, b1d.reshape(1, h), W2d, b2d.reshape(1, s_dim))


def _tc_unpack(acc, T):
  def body(a_ref, o_ref):
    mx = jnp.max(a_ref[...], axis=0, keepdims=True)
    o_ref[...] = jnp.int32(2047) - (mx & jnp.int32(2047))

  return pl.pallas_call(
      body,
      out_shape=jax.ShapeDtypeStruct((1, T), jnp.int32),
  )(acc)


def _tc_final(msds, big, bce, b_real, s_dim):
  BP = msds.shape[0]
  BLK = 4096
  G = BP // BLK

  def body(a_ref, g_ref, bce_ref, out_ref):
    i = pl.program_id(0)
    d = a_ref[...] - g_ref[...]
    ridx = i * BLK + lax.broadcasted_iota(jnp.int32, (BLK, 1), 0)
    s = jnp.sum(jnp.where(ridx < b_real, d * d, 0.0), axis=(0, 1),
                keepdims=True)
    acc = jnp.where(i == 0, s, out_ref[...] + s)
    out_ref[...] = jnp.where(
        i == G - 1,
        acc / (2.0 * b_real * s_dim) + bce_ref[...] / b_real,
        acc)

  return pl.pallas_call(
      body,
      grid=(G,),
      in_specs=[
          pl.BlockSpec((BLK, 2 * s_dim), lambda i: (i, 0)),
          pl.BlockSpec((BLK, 2 * s_dim), lambda i: (i, 0)),
          pl.BlockSpec((1, 1), lambda i: (0, 0)),
      ],
      out_specs=pl.BlockSpec((1, 1), lambda i: (0, 0)),
      out_shape=jax.ShapeDtypeStruct((1, 1), jnp.float32),
  )(msds, big, bce)


def kernel(miRNA_embeddings, disease_embeddings, miRNA_index, disease_index,
           true_labels, W_m, b_m, W_d, b_d, W_ms1, b_ms1, W_ms2, b_ms2,
           W_ds1, b_ds1, W_ds2, b_ds2):
  num_m = miRNA_embeddings.shape[0]
  num_d = disease_embeddings.shape[0]
  B = miRNA_index.shape[0]
  s_dim = W_ms2.shape[0]

  BP = 16384 * (-(-B // 16384))
  steps = BP // (NW * IDXW)
  pad = BP - B
  mi = miRNA_index.astype(jnp.int32)
  di = disease_index.astype(jnp.int32)
  mi_g = jnp.pad(mi, (0, pad))
  di_g = jnp.pad(di, (0, pad))
  mi_s = jnp.pad(mi, (0, pad), constant_values=num_m)
  lab = jnp.pad(true_labels, (0, pad)).reshape(BP, 1)

  mi2 = mi_g.reshape(NW, steps, IDXW)
  di2 = di_g.reshape(NW, steps, IDXW)

  me = _sc_gather(miRNA_embeddings, mi2, steps)
  de = _sc_gather(disease_embeddings, di2, steps)

  pay, msds, bce = _tc_mlp(me, de, lab, W_m, b_m, W_d, b_d,
                           W_ms1, b_ms1, W_ms2, b_ms2,
                           W_ds1, b_ds1, W_ds2, b_ds2, B)

  bi_n = NW * IDXW * (-(-num_m // (NW * IDXW)))
  bi_steps = bi_n // (NW * IDXW)
  acc = _sc_segmax(pay.reshape(NW, steps, IDXW),
                   mi_s.reshape(NW, steps, IDXW), di2, steps, bi_n)
  bi2 = _tc_unpack(acc, bi_n)

  bi_pad = bi2.reshape(NW, bi_steps, IDXW)
  g2 = _sc_gather(msds, bi_pad, bi_steps)
  big = _sc_gather(g2, mi2, steps)

  loss = _tc_final(msds, big, bce, B, s_dim)
  return (pay.reshape(-1)[:B], loss[0, 0])


# speedup vs baseline: 2.8253x; 1.0838x over previous
import functools

import jax
import jax.numpy as jnp
from jax import lax
from jax.experimental import pallas as pl
from jax.experimental.pallas import tpu as pltpu
from jax.experimental.pallas import tpu_sc as plsc

NC, NS = 2, 16
NW = NC * NS
IDXW = 128


def _sc_mesh():
  return plsc.VectorSubcoreMesh(
      core_axis_name="c", subcore_axis_name="s",
      num_cores=NC, num_subcores=NS)


def _sc_gather(table, idx3, steps):
  _, D = table.shape
  N = NW * steps * IDXW

  @functools.partial(
      pl.kernel,
      out_type=jax.ShapeDtypeStruct((N, D), jnp.float32),
      mesh=_sc_mesh(),
      compiler_params=pltpu.CompilerParams(use_tc_tiling_on_sc=False),
      scratch_types=[
          pltpu.VMEM((steps, IDXW), jnp.int32),
          pltpu.VMEM((IDXW, D), jnp.float32),
          pltpu.SemaphoreType.DMA,
      ],
  )
  def k(table_hbm, idx_hbm, out_hbm, idx_v, rows_v, sem):
    w = lax.axis_index("s") * NC + lax.axis_index("c")
    pltpu.sync_copy(idx_hbm.at[w], idx_v)

    def body(s, carry):
      pltpu.async_copy(table_hbm.at[idx_v.at[s]], rows_v, sem).wait()
      pltpu.sync_copy(rows_v, out_hbm.at[pl.ds((w * steps + s) * IDXW, IDXW)])
      return carry

    lax.fori_loop(0, steps, body, 0)

  return k(table, idx3)


def _sc_gather2(tab_a, tab_b, idx_a3, idx_b3, steps):
  Da = tab_a.shape[1]
  Db = tab_b.shape[1]
  N = NW * steps * IDXW

  @functools.partial(
      pl.kernel,
      out_type=jax.ShapeDtypeStruct((N, Da + Db), jnp.float32),
      mesh=_sc_mesh(),
      compiler_params=pltpu.CompilerParams(use_tc_tiling_on_sc=False),
      scratch_types=[
          pltpu.VMEM((steps, IDXW), jnp.int32),
          pltpu.VMEM((steps, IDXW), jnp.int32),
          pltpu.VMEM((IDXW, Da), jnp.float32),
          pltpu.VMEM((IDXW, Db), jnp.float32),
          pltpu.SemaphoreType.DMA,
          pltpu.SemaphoreType.DMA,
      ],
  )
  def k(ta_hbm, tb_hbm, ia_hbm, ib_hbm, out_hbm, ia_v, ib_v, ra_v, rb_v,
        sa, sb):
    w = lax.axis_index("s") * NC + lax.axis_index("c")
    pltpu.sync_copy(ia_hbm.at[w], ia_v)
    pltpu.sync_copy(ib_hbm.at[w], ib_v)

    def body(s, carry):
      r0 = (w * steps + s) * IDXW
      ha = pltpu.async_copy(ta_hbm.at[ia_v.at[s]], ra_v, sa)
      hb = pltpu.async_copy(tb_hbm.at[ib_v.at[s]], rb_v, sb)
      ha.wait()
      pltpu.sync_copy(ra_v, out_hbm.at[pl.ds(r0, IDXW), pl.ds(0, Da)])
      hb.wait()
      pltpu.sync_copy(rb_v, out_hbm.at[pl.ds(r0, IDXW), pl.ds(Da, Db)])
      return carry

    lax.fori_loop(0, steps, body, 0)

  return k(tab_a, tab_b, idx_a3, idx_b3)


def _sc_segmax(vals3, mi3, di3, steps, T):
  seg = T // NS

  @functools.partial(
      pl.kernel,
      out_type=jax.ShapeDtypeStruct((NC, T), jnp.int32),
      mesh=_sc_mesh(),
      compiler_params=pltpu.CompilerParams(use_tc_tiling_on_sc=False,
                                           needs_layout_passes=False),
      scratch_types=[
          pltpu.VMEM((steps, IDXW), jnp.float32),
          pltpu.VMEM((steps, IDXW), jnp.int32),
          pltpu.VMEM((steps, IDXW), jnp.int32),
          pltpu.VMEM((T,), jnp.int32),
          pltpu.VMEM((seg,), jnp.int32),
          pltpu.VMEM((seg,), jnp.int32),
          pltpu.VMEM_SHARED((NS, T), jnp.int32),
      ],
  )
  def k(val_hbm, mi_hbm, di_hbm, out_hbm, val_v, mi_v, di_v, tbl, mseg,
        tseg, spm):
    c = lax.axis_index("c")
    sid = lax.axis_index("s")
    w = sid * NC + c

    def init(i, carry):
      tbl[pl.ds(i * 16, 16)] = jnp.full((16,), 2047, jnp.int32)
      return carry

    lax.fori_loop(0, T // 16, init, 0)
    pltpu.sync_copy(val_hbm.at[w], val_v)
    pltpu.sync_copy(mi_hbm.at[w], mi_v)
    pltpu.sync_copy(di_hbm.at[w], di_v)

    def row(s, carry):
      def vec(kk, carry2):
        m = mi_v[s, pl.ds(kk * 16, 16)]
        d = di_v[s, pl.ds(kk * 16, 16)]
        v = val_v[s, pl.ds(kk * 16, 16)]
        b = plsc.bitcast(v, jnp.int32)
        key32 = jnp.where(b >= 0, b, b ^ jnp.int32(0x7FFFFFFF))
        key = (key32 & jnp.int32(-2048)) | (jnp.int32(2047) - d)

        def w_cond(mask):
          return jnp.max(jnp.where(mask, 1, 0)) > 0

        def w_body(mask):
          plsc.store_scatter(tbl, [m], key, mask=mask)
          cur = plsc.load_gather(tbl, [m])
          return mask & (cur < key)

        lax.while_loop(w_cond, w_body,
                       key > plsc.load_gather(tbl, [m]))
        return carry2

      lax.fori_loop(0, IDXW // 16, vec, carry)
      return carry

    lax.fori_loop(0, steps, row, 0)

    pltpu.sync_copy(tbl, spm.at[sid])
    plsc.subcore_barrier()
    pltpu.sync_copy(spm.at[0, pl.ds(sid * seg, seg)], mseg)

    def merge(j, carry):
      pltpu.sync_copy(spm.at[j, pl.ds(sid * seg, seg)], tseg)

      def mvec(i, carry2):
        mseg[pl.ds(i * 16, 16)] = jnp.maximum(
            mseg[pl.ds(i * 16, 16)], tseg[pl.ds(i * 16, 16)])
        return carry2

      lax.fori_loop(0, seg // 16, mvec, carry)
      return carry

    lax.fori_loop(1, NS, merge, 0)
    pltpu.sync_copy(mseg, out_hbm.at[c, pl.ds(sid * seg, seg)])

  return k(vals3, mi3, di3)


def _tc_mlp(medde, lab, Wa, ba, Wb, bb, Wc, bc, b_real):
  BP = medde.shape[0]
  BLK = 8192
  G = BP // BLK
  s2 = Wc.shape[0]
  s_dim = s2 // 2

  def body(x_ref, lab_ref, Wa_r, ba_r, Wb_r, bb_r, Wc_r, bc_r,
           pay_ref, msds_ref, bce_ref):
    i = pl.program_id(0)

    def dn(x, w_ref):
      return lax.dot_general(x, w_ref[...], (((1,), (1,)), ((), ())),
                             preferred_element_type=jnp.float32,
                             precision=lax.Precision.HIGHEST)

    pmd = dn(x_ref[...], Wa_r) + ba_r[...]
    msds = dn(jnp.maximum(dn(pmd, Wb_r) + bb_r[...], 0.0), Wc_r) + bc_r[...]
    ms = msds[:, :s_dim]
    ds = msds[:, s_dim:]
    num = jnp.sum(ms * ds, axis=1, keepdims=True)
    den = jnp.sqrt(jnp.sum(ms * ms, axis=1, keepdims=True)) * \
        jnp.sqrt(jnp.sum(ds * ds, axis=1, keepdims=True))
    x = num / den
    pay_ref[...] = x
    msds_ref[...] = msds
    y = lab_ref[...]
    bce = jnp.maximum(x, 0.0) - x * y + jnp.log1p(jnp.exp(-jnp.abs(x)))
    ridx = i * BLK + lax.broadcasted_iota(jnp.int32, (BLK, 1), 0)
    s = jnp.sum(jnp.where(ridx < b_real, bce, 0.0), axis=(0, 1),
                keepdims=True)
    bce_ref[...] = jnp.where(i == 0, s, bce_ref[...] + s)

  rep = lambda shape: pl.BlockSpec(shape, lambda i: (0, 0))
  return pl.pallas_call(
      body,
      grid=(G,),
      in_specs=[
          pl.BlockSpec((BLK, medde.shape[1]), lambda i: (i, 0)),
          pl.BlockSpec((BLK, 1), lambda i: (i, 0)),
          rep(Wa.shape), rep(ba.shape), rep(Wb.shape), rep(bb.shape),
          rep(Wc.shape), rep(bc.shape),
      ],
      out_specs=[
          pl.BlockSpec((BLK, 1), lambda i: (i, 0)),
          pl.BlockSpec((BLK, s2), lambda i: (i, 0)),
          pl.BlockSpec((1, 1), lambda i: (0, 0)),
      ],
      out_shape=[
          jax.ShapeDtypeStruct((BP, 1), jnp.float32),
          jax.ShapeDtypeStruct((BP, s2), jnp.float32),
          jax.ShapeDtypeStruct((1, 1), jnp.float32),
      ],
  )(medde, lab, Wa, ba, Wb, bb, Wc, bc)


def _tc_unpack(acc, T):
  def body(a_ref, o_ref):
    mx = jnp.max(a_ref[...], axis=0, keepdims=True)
    o_ref[...] = jnp.int32(2047) - (mx & jnp.int32(2047))

  return pl.pallas_call(
      body,
      out_shape=jax.ShapeDtypeStruct((1, T), jnp.int32),
  )(acc)


def _tc_final(msds, big, bce, b_real, s_dim):
  BP = msds.shape[0]
  BLK = 4096
  G = BP // BLK

  def body(a_ref, g_ref, bce_ref, out_ref):
    i = pl.program_id(0)
    d = a_ref[...] - g_ref[...]
    ridx = i * BLK + lax.broadcasted_iota(jnp.int32, (BLK, 1), 0)
    s = jnp.sum(jnp.where(ridx < b_real, d * d, 0.0), axis=(0, 1),
                keepdims=True)
    acc = jnp.where(i == 0, s, out_ref[...] + s)
    out_ref[...] = jnp.where(
        i == G - 1,
        acc / (2.0 * b_real * s_dim) + bce_ref[...] / b_real,
        acc)

  return pl.pallas_call(
      body,
      grid=(G,),
      in_specs=[
          pl.BlockSpec((BLK, 2 * s_dim), lambda i: (i, 0)),
          pl.BlockSpec((BLK, 2 * s_dim), lambda i: (i, 0)),
          pl.BlockSpec((1, 1), lambda i: (0, 0)),
      ],
      out_specs=pl.BlockSpec((1, 1), lambda i: (0, 0)),
      out_shape=jax.ShapeDtypeStruct((1, 1), jnp.float32),
  )(msds, big, bce)


def kernel(miRNA_embeddings, disease_embeddings, miRNA_index, disease_index,
           true_labels, W_m, b_m, W_d, b_d, W_ms1, b_ms1, W_ms2, b_ms2,
           W_ds1, b_ds1, W_ds2, b_ds2):
  num_m = miRNA_embeddings.shape[0]
  num_d = disease_embeddings.shape[0]
  B = miRNA_index.shape[0]
  s_dim = W_ms2.shape[0]

  BP = 16384 * (-(-B // 16384))
  steps = BP // (NW * IDXW)
  pad = BP - B
  mi = miRNA_index.astype(jnp.int32)
  di = disease_index.astype(jnp.int32)
  mi_g = jnp.pad(mi, (0, pad))
  di_g = jnp.pad(di, (0, pad))
  mi_s = jnp.pad(mi, (0, pad), constant_values=num_m)
  lab = jnp.pad(true_labels, (0, pad)).reshape(BP, 1)

  mi2 = mi_g.reshape(NW, steps, IDXW)
  di2 = di_g.reshape(NW, steps, IDXW)

  medde = _sc_gather2(miRNA_embeddings, disease_embeddings, mi2, di2, steps)

  h = W_m.shape[0]
  fm = W_m.shape[1]
  fd = W_d.shape[1]
  s_dim = W_ms2.shape[0]
  Wa = jnp.zeros((2 * h, fm + fd), jnp.float32)
  Wa = Wa.at[:h, :fm].set(W_m).at[h:, fm:].set(W_d)
  ba = jnp.concatenate([b_m, b_d]).reshape(1, 2 * h)
  Wb = jnp.zeros((2 * h, 2 * h), jnp.float32)
  Wb = Wb.at[:h, :h].set(W_ms1).at[h:, h:].set(W_ds1)
  bb = jnp.concatenate([b_ms1, b_ds1]).reshape(1, 2 * h)
  Wc = jnp.zeros((2 * s_dim, 2 * h), jnp.float32)
  Wc = Wc.at[:s_dim, :h].set(W_ms2).at[s_dim:, h:].set(W_ds2)
  bc = jnp.concatenate([b_ms2, b_ds2]).reshape(1, 2 * s_dim)

  pay, msds, bce = _tc_mlp(medde, lab, Wa, ba, Wb, bb, Wc, bc, B)

  bi_n = NW * IDXW * (-(-num_m // (NW * IDXW)))
  bi_steps = bi_n // (NW * IDXW)
  acc = _sc_segmax(pay.reshape(NW, steps, IDXW),
                   mi_s.reshape(NW, steps, IDXW), di2, steps, bi_n)
  bi2 = _tc_unpack(acc, bi_n)

  bi_pad = bi2.reshape(NW, bi_steps, IDXW)
  g2 = _sc_gather(msds, bi_pad, bi_steps)
  big = _sc_gather(g2, mi2, steps)

  loss = _tc_final(msds, big, bce, B, s_dim)
  return (pay.reshape(-1)[:B], loss[0, 0])


# speedup vs baseline: 7.5318x; 2.6658x over previous
import functools

import jax
import jax.numpy as jnp
from jax import lax
from jax.experimental import pallas as pl
from jax.experimental.pallas import tpu as pltpu
from jax.experimental.pallas import tpu_sc as plsc

NC, NS = 2, 16
NW = NC * NS
IDXW = 128


def _sc_mesh():
  return plsc.VectorSubcoreMesh(
      core_axis_name="c", subcore_axis_name="s",
      num_cores=NC, num_subcores=NS)


def _sc_gather(table, idx3, steps):
  _, D = table.shape
  N = NW * steps * IDXW

  @functools.partial(
      pl.kernel,
      out_type=jax.ShapeDtypeStruct((N, D), jnp.float32),
      mesh=_sc_mesh(),
      compiler_params=pltpu.CompilerParams(use_tc_tiling_on_sc=False),
      scratch_types=[
          pltpu.VMEM((steps, IDXW), jnp.int32),
          pltpu.VMEM((IDXW, D), jnp.float32),
          pltpu.SemaphoreType.DMA,
      ],
  )
  def k(table_hbm, idx_hbm, out_hbm, idx_v, rows_v, sem):
    w = lax.axis_index("s") * NC + lax.axis_index("c")
    pltpu.sync_copy(idx_hbm.at[w], idx_v)

    def body(s, carry):
      pltpu.async_copy(table_hbm.at[idx_v.at[s]], rows_v, sem).wait()
      pltpu.sync_copy(rows_v, out_hbm.at[pl.ds((w * steps + s) * IDXW, IDXW)])
      return carry

    lax.fori_loop(0, steps, body, 0)

  return k(table, idx3)


def _sc_gather2(tab_a, tab_b, idx_a3, idx_b3, steps):
  Da = tab_a.shape[1]
  Db = tab_b.shape[1]
  N = NW * steps * IDXW

  @functools.partial(
      pl.kernel,
      out_type=jax.ShapeDtypeStruct((N, Da + Db), jnp.float32),
      mesh=_sc_mesh(),
      compiler_params=pltpu.CompilerParams(use_tc_tiling_on_sc=False),
      scratch_types=[
          pltpu.VMEM((steps, IDXW), jnp.int32),
          pltpu.VMEM((steps, IDXW), jnp.int32),
          pltpu.VMEM((IDXW, Da), jnp.float32),
          pltpu.VMEM((IDXW, Db), jnp.float32),
          pltpu.SemaphoreType.DMA,
          pltpu.SemaphoreType.DMA,
      ],
  )
  def k(ta_hbm, tb_hbm, ia_hbm, ib_hbm, out_hbm, ia_v, ib_v, ra_v, rb_v,
        sa, sb):
    w = lax.axis_index("s") * NC + lax.axis_index("c")
    pltpu.sync_copy(ia_hbm.at[w], ia_v)
    pltpu.sync_copy(ib_hbm.at[w], ib_v)

    def body(s, carry):
      r0 = (w * steps + s) * IDXW
      ha = pltpu.async_copy(ta_hbm.at[ia_v.at[s]], ra_v, sa)
      hb = pltpu.async_copy(tb_hbm.at[ib_v.at[s]], rb_v, sb)
      ha.wait()
      pltpu.sync_copy(ra_v, out_hbm.at[pl.ds(r0, IDXW), pl.ds(0, Da)])
      hb.wait()
      pltpu.sync_copy(rb_v, out_hbm.at[pl.ds(r0, IDXW), pl.ds(Da, Db)])
      return carry

    lax.fori_loop(0, steps, body, 0)

  return k(tab_a, tab_b, idx_a3, idx_b3)


def _sc_segmax(vals3, mi3, di3, steps, T):
  seg = T // NS

  @functools.partial(
      pl.kernel,
      out_type=jax.ShapeDtypeStruct((NC, T), jnp.int32),
      mesh=_sc_mesh(),
      compiler_params=pltpu.CompilerParams(use_tc_tiling_on_sc=False,
                                           needs_layout_passes=False),
      scratch_types=[
          pltpu.VMEM((steps, IDXW), jnp.float32),
          pltpu.VMEM((steps, IDXW), jnp.int32),
          pltpu.VMEM((steps, IDXW), jnp.int32),
          pltpu.VMEM((T,), jnp.int32),
          pltpu.VMEM((seg,), jnp.int32),
          pltpu.VMEM((seg,), jnp.int32),
          pltpu.VMEM_SHARED((NS, T), jnp.int32),
      ],
  )
  def k(val_hbm, mi_hbm, di_hbm, out_hbm, val_v, mi_v, di_v, tbl, mseg,
        tseg, spm):
    c = lax.axis_index("c")
    sid = lax.axis_index("s")
    w = sid * NC + c

    def init(i, carry):
      tbl[pl.ds(i * 16, 16)] = jnp.full((16,), 2047, jnp.int32)
      return carry

    lax.fori_loop(0, T // 16, init, 0)
    pltpu.sync_copy(val_hbm.at[w], val_v)
    pltpu.sync_copy(mi_hbm.at[w], mi_v)
    pltpu.sync_copy(di_hbm.at[w], di_v)

    def row(s, carry):
      def vec(kk, carry2):
        m = mi_v[s, pl.ds(kk * 16, 16)]
        d = di_v[s, pl.ds(kk * 16, 16)]
        v = val_v[s, pl.ds(kk * 16, 16)]
        b = plsc.bitcast(v, jnp.int32)
        key32 = jnp.where(b >= 0, b, b ^ jnp.int32(0x7FFFFFFF))
        key = (key32 & jnp.int32(-2048)) | (jnp.int32(2047) - d)

        def w_cond(mask):
          return jnp.max(jnp.where(mask, 1, 0)) > 0

        def w_body(mask):
          plsc.store_scatter(tbl, [m], key, mask=mask)
          cur = plsc.load_gather(tbl, [m])
          return mask & (cur < key)

        lax.while_loop(w_cond, w_body,
                       key > plsc.load_gather(tbl, [m]))
        return carry2

      lax.fori_loop(0, IDXW // 16, vec, carry)
      return carry

    lax.fori_loop(0, steps, row, 0)

    pltpu.sync_copy(tbl, spm.at[sid])
    plsc.subcore_barrier()
    pltpu.sync_copy(spm.at[0, pl.ds(sid * seg, seg)], mseg)

    def merge(j, carry):
      pltpu.sync_copy(spm.at[j, pl.ds(sid * seg, seg)], tseg)

      def mvec(i, carry2):
        mseg[pl.ds(i * 16, 16)] = jnp.maximum(
            mseg[pl.ds(i * 16, 16)], tseg[pl.ds(i * 16, 16)])
        return carry2

      lax.fori_loop(0, seg // 16, mvec, carry)
      return carry

    lax.fori_loop(1, NS, merge, 0)
    pltpu.sync_copy(mseg, out_hbm.at[c, pl.ds(sid * seg, seg)])

  return k(vals3, mi3, di3)


def _tc_mlp(medde, lab, Wa, ba, Wb, bb, Wc, bc, b_real):
  BP = medde.shape[0]
  BLK = 8192
  G = BP // BLK
  s2 = Wc.shape[0]
  s_dim = s2 // 2

  def body(x_ref, lab_ref, Wa_r, ba_r, Wb_r, bb_r, Wc_r, bc_r,
           pay_ref, msds_ref, bce_ref):
    i = pl.program_id(0)

    def dnT(w_ref, xT, dims):
      return lax.dot_general(w_ref[...], xT, (dims, ((), ())),
                             preferred_element_type=jnp.float32)

    pmdT = dnT(Wa_r, x_ref[...], ((1,), (1,))) + ba_r[...]
    hT = jnp.maximum(dnT(Wb_r, pmdT, ((1,), (0,))) + bb_r[...], 0.0)
    msdsT = dnT(Wc_r, hT, ((1,), (0,))) + bc_r[...]
    msT = msdsT[:s_dim, :]
    dsT = msdsT[s_dim:, :]
    num = jnp.sum(msT * dsT, axis=0, keepdims=True)
    den = jnp.sqrt(jnp.sum(msT * msT, axis=0, keepdims=True)) * \
        jnp.sqrt(jnp.sum(dsT * dsT, axis=0, keepdims=True))
    x = num / den
    pay_ref[...] = x[None]
    msds_ref[...] = msdsT.T
    y = lab_ref[0]
    bce = jnp.maximum(x, 0.0) - x * y + jnp.log1p(jnp.exp(-jnp.abs(x)))
    cidx = i * BLK + lax.broadcasted_iota(jnp.int32, (1, BLK), 1)
    s = jnp.sum(jnp.where(cidx < b_real, bce, 0.0), axis=(0, 1),
                keepdims=True)
    bce_ref[...] = jnp.where(i == 0, s, bce_ref[...] + s)

  rep = lambda shape: pl.BlockSpec(shape, lambda i: tuple(0 for _ in shape))
  return pl.pallas_call(
      body,
      grid=(G,),
      in_specs=[
          pl.BlockSpec((BLK, medde.shape[1]), lambda i: (i, 0)),
          pl.BlockSpec((1, 1, BLK), lambda i: (i, 0, 0)),
          rep(Wa.shape), rep(ba.shape), rep(Wb.shape), rep(bb.shape),
          rep(Wc.shape), rep(bc.shape),
      ],
      out_specs=[
          pl.BlockSpec((1, 1, BLK), lambda i: (i, 0, 0)),
          pl.BlockSpec((BLK, s2), lambda i: (i, 0)),
          pl.BlockSpec((1, 1), lambda i: (0, 0)),
      ],
      out_shape=[
          jax.ShapeDtypeStruct((G, 1, BLK), jnp.float32),
          jax.ShapeDtypeStruct((BP, s2), jnp.float32),
          jax.ShapeDtypeStruct((1, 1), jnp.float32),
      ],
  )(medde, lab, Wa, ba, Wb, bb, Wc, bc)


def _tc_unpack(acc, T):
  def body(a_ref, o_ref):
    mx = jnp.max(a_ref[...], axis=0, keepdims=True)
    o_ref[...] = jnp.int32(2047) - (mx & jnp.int32(2047))

  return pl.pallas_call(
      body,
      out_shape=jax.ShapeDtypeStruct((1, T), jnp.int32),
  )(acc)


def _tc_final(msds, big, bce, b_real, s_dim):
  BP = msds.shape[0]
  BLK = 4096
  G = BP // BLK

  def body(a_ref, g_ref, bce_ref, out_ref):
    i = pl.program_id(0)
    d = a_ref[...] - g_ref[...]
    ridx = i * BLK + lax.broadcasted_iota(jnp.int32, (BLK, 1), 0)
    s = jnp.sum(jnp.where(ridx < b_real, d * d, 0.0), axis=(0, 1),
                keepdims=True)
    acc = jnp.where(i == 0, s, out_ref[...] + s)
    out_ref[...] = jnp.where(
        i == G - 1,
        acc / (2.0 * b_real * s_dim) + bce_ref[...] / b_real,
        acc)

  return pl.pallas_call(
      body,
      grid=(G,),
      in_specs=[
          pl.BlockSpec((BLK, 2 * s_dim), lambda i: (i, 0)),
          pl.BlockSpec((BLK, 2 * s_dim), lambda i: (i, 0)),
          pl.BlockSpec((1, 1), lambda i: (0, 0)),
      ],
      out_specs=pl.BlockSpec((1, 1), lambda i: (0, 0)),
      out_shape=jax.ShapeDtypeStruct((1, 1), jnp.float32),
  )(msds, big, bce)


def kernel(miRNA_embeddings, disease_embeddings, miRNA_index, disease_index,
           true_labels, W_m, b_m, W_d, b_d, W_ms1, b_ms1, W_ms2, b_ms2,
           W_ds1, b_ds1, W_ds2, b_ds2):
  num_m = miRNA_embeddings.shape[0]
  num_d = disease_embeddings.shape[0]
  B = miRNA_index.shape[0]
  s_dim = W_ms2.shape[0]

  BP = 16384 * (-(-B // 16384))
  steps = BP // (NW * IDXW)
  pad = BP - B
  mi = miRNA_index.astype(jnp.int32)
  di = disease_index.astype(jnp.int32)
  mi_g = jnp.pad(mi, (0, pad))
  di_g = jnp.pad(di, (0, pad))
  mi_s = jnp.pad(mi, (0, pad), constant_values=num_m)
  lab = jnp.pad(true_labels, (0, pad)).reshape(BP // 8192, 1, 8192)

  mi2 = mi_g.reshape(NW, steps, IDXW)
  di2 = di_g.reshape(NW, steps, IDXW)

  medde = _sc_gather2(miRNA_embeddings, disease_embeddings, mi2, di2, steps)

  h = W_m.shape[0]
  fm = W_m.shape[1]
  fd = W_d.shape[1]
  s_dim = W_ms2.shape[0]
  Wa = jnp.zeros((2 * h, fm + fd), jnp.float32)
  Wa = Wa.at[:h, :fm].set(W_m).at[h:, fm:].set(W_d)
  ba = jnp.concatenate([b_m, b_d]).reshape(2 * h, 1)
  Wb = jnp.zeros((2 * h, 2 * h), jnp.float32)
  Wb = Wb.at[:h, :h].set(W_ms1).at[h:, h:].set(W_ds1)
  bb = jnp.concatenate([b_ms1, b_ds1]).reshape(2 * h, 1)
  Wc = jnp.zeros((2 * s_dim, 2 * h), jnp.float32)
  Wc = Wc.at[:s_dim, :h].set(W_ms2).at[s_dim:, h:].set(W_ds2)
  bc = jnp.concatenate([b_ms2, b_ds2]).reshape(2 * s_dim, 1)

  pay, msds, bce = _tc_mlp(medde, lab, Wa, ba, Wb, bb, Wc, bc, B)

  bi_n = NW * IDXW * (-(-num_m // (NW * IDXW)))
  bi_steps = bi_n // (NW * IDXW)
  acc = _sc_segmax(pay.reshape(NW, steps, IDXW),
                   mi_s.reshape(NW, steps, IDXW), di2, steps, bi_n)
  bi2 = _tc_unpack(acc, bi_n)

  bi_pad = bi2.reshape(NW, bi_steps, IDXW)
  g2 = _sc_gather(msds, bi_pad, bi_steps)
  big = _sc_gather(g2, mi2, steps)

  loss = _tc_final(msds, big, bce, B, s_dim)
  return (pay.reshape(-1)[:B], loss[0, 0])


# speedup vs baseline: 8.1614x; 1.0836x over previous
import functools

import jax
import jax.numpy as jnp
from jax import lax
from jax.experimental import pallas as pl
from jax.experimental.pallas import tpu as pltpu
from jax.experimental.pallas import tpu_sc as plsc

NC, NS = 2, 16
NW = NC * NS
IDXW = 128


def _sc_mesh():
  return plsc.VectorSubcoreMesh(
      core_axis_name="c", subcore_axis_name="s",
      num_cores=NC, num_subcores=NS)


def _sc_gather(table, idx3, steps):
  _, D = table.shape
  N = NW * steps * IDXW

  @functools.partial(
      pl.kernel,
      out_type=jax.ShapeDtypeStruct((N, D), jnp.float32),
      mesh=_sc_mesh(),
      compiler_params=pltpu.CompilerParams(use_tc_tiling_on_sc=False),
      scratch_types=[
          pltpu.VMEM((steps, IDXW), jnp.int32),
          pltpu.VMEM((IDXW, D), jnp.float32),
          pltpu.SemaphoreType.DMA,
      ],
  )
  def k(table_hbm, idx_hbm, out_hbm, idx_v, rows_v, sem):
    w = lax.axis_index("s") * NC + lax.axis_index("c")
    pltpu.sync_copy(idx_hbm.at[w], idx_v)

    def body(s, carry):
      pltpu.async_copy(table_hbm.at[idx_v.at[s]], rows_v, sem).wait()
      pltpu.sync_copy(rows_v, out_hbm.at[pl.ds((w * steps + s) * IDXW, IDXW)])
      return carry

    lax.fori_loop(0, steps, body, 0)

  return k(table, idx3)


def _sc_gatherp(table, idx3, steps):
  _, D = table.shape
  N = NW * steps * IDXW

  @functools.partial(
      pl.kernel,
      out_type=jax.ShapeDtypeStruct((N, D), jnp.float32),
      mesh=_sc_mesh(),
      compiler_params=pltpu.CompilerParams(use_tc_tiling_on_sc=False),
      scratch_types=[
          pltpu.VMEM((steps, IDXW), jnp.int32),
          pltpu.VMEM((IDXW, D), jnp.float32),
          pltpu.VMEM((IDXW, D), jnp.float32),
          pltpu.SemaphoreType.DMA,
          pltpu.SemaphoreType.DMA,
      ],
  )
  def k(t_hbm, idx_hbm, out_hbm, idx_v, r0, r1, s0, s1):
    w = lax.axis_index("s") * NC + lax.axis_index("c")
    base = w * steps
    pltpu.sync_copy(idx_hbm.at[w], idx_v)
    pltpu.async_copy(t_hbm.at[idx_v.at[0]], r0, s0)

    def body(t, carry):
      sa = 2 * t
      sb = 2 * t + 1
      pltpu.async_copy(t_hbm.at[idx_v.at[sb]], r1, s1)
      pltpu.make_async_copy(t_hbm.at[idx_v.at[sa]], r0, s0).wait()
      pltpu.sync_copy(r0, out_hbm.at[pl.ds((base + sa) * IDXW, IDXW)])

      @pl.when(sa + 2 < steps)
      def _():
        pltpu.async_copy(t_hbm.at[idx_v.at[sa + 2]], r0, s0)

      pltpu.make_async_copy(t_hbm.at[idx_v.at[sb]], r1, s1).wait()
      pltpu.sync_copy(r1, out_hbm.at[pl.ds((base + sb) * IDXW, IDXW)])
      return carry

    lax.fori_loop(0, steps // 2, body, 0)

  return k(table, idx3)


def _sc_gather2(tab_a, tab_b, idx_a3, idx_b3, steps):
  Da = tab_a.shape[1]
  Db = tab_b.shape[1]
  N = NW * steps * IDXW

  @functools.partial(
      pl.kernel,
      out_type=jax.ShapeDtypeStruct((N, Da + Db), jnp.float32),
      mesh=_sc_mesh(),
      compiler_params=pltpu.CompilerParams(use_tc_tiling_on_sc=False),
      scratch_types=[
          pltpu.VMEM((steps, IDXW), jnp.int32),
          pltpu.VMEM((steps, IDXW), jnp.int32),
          pltpu.VMEM((IDXW, Da), jnp.float32),
          pltpu.VMEM((IDXW, Da), jnp.float32),
          pltpu.VMEM((IDXW, Db), jnp.float32),
          pltpu.VMEM((IDXW, Db), jnp.float32),
          pltpu.SemaphoreType.DMA,
          pltpu.SemaphoreType.DMA,
          pltpu.SemaphoreType.DMA,
          pltpu.SemaphoreType.DMA,
      ],
  )
  def k(ta_hbm, tb_hbm, ia_hbm, ib_hbm, out_hbm, ia_v, ib_v, ra0, ra1,
        rb0, rb1, sa0, sa1, sb0, sb1):
    w = lax.axis_index("s") * NC + lax.axis_index("c")
    base = w * steps
    pltpu.sync_copy(ia_hbm.at[w], ia_v)
    pltpu.sync_copy(ib_hbm.at[w], ib_v)
    pltpu.async_copy(ta_hbm.at[ia_v.at[0]], ra0, sa0)
    pltpu.async_copy(tb_hbm.at[ib_v.at[0]], rb0, sb0)

    def body(t, carry):
      sa = 2 * t
      sb = 2 * t + 1
      pltpu.async_copy(ta_hbm.at[ia_v.at[sb]], ra1, sa1)
      pltpu.async_copy(tb_hbm.at[ib_v.at[sb]], rb1, sb1)
      r0 = (base + sa) * IDXW
      pltpu.make_async_copy(ta_hbm.at[ia_v.at[sa]], ra0, sa0).wait()
      pltpu.sync_copy(ra0, out_hbm.at[pl.ds(r0, IDXW), pl.ds(0, Da)])
      pltpu.make_async_copy(tb_hbm.at[ib_v.at[sa]], rb0, sb0).wait()
      pltpu.sync_copy(rb0, out_hbm.at[pl.ds(r0, IDXW), pl.ds(Da, Db)])

      @pl.when(sa + 2 < steps)
      def _():
        pltpu.async_copy(ta_hbm.at[ia_v.at[sa + 2]], ra0, sa0)
        pltpu.async_copy(tb_hbm.at[ib_v.at[sa + 2]], rb0, sb0)

      r1 = (base + sb) * IDXW
      pltpu.make_async_copy(ta_hbm.at[ia_v.at[sb]], ra1, sa1).wait()
      pltpu.sync_copy(ra1, out_hbm.at[pl.ds(r1, IDXW), pl.ds(0, Da)])
      pltpu.make_async_copy(tb_hbm.at[ib_v.at[sb]], rb1, sb1).wait()
      pltpu.sync_copy(rb1, out_hbm.at[pl.ds(r1, IDXW), pl.ds(Da, Db)])
      return carry

    lax.fori_loop(0, steps // 2, body, 0)

  return k(tab_a, tab_b, idx_a3, idx_b3)


def _sc_segmax(vals3, mi3, di3, steps, T):
  seg = T // NS

  @functools.partial(
      pl.kernel,
      out_type=jax.ShapeDtypeStruct((NC, T), jnp.int32),
      mesh=_sc_mesh(),
      compiler_params=pltpu.CompilerParams(use_tc_tiling_on_sc=False,
                                           needs_layout_passes=False),
      scratch_types=[
          pltpu.VMEM((steps, IDXW), jnp.float32),
          pltpu.VMEM((steps, IDXW), jnp.int32),
          pltpu.VMEM((steps, IDXW), jnp.int32),
          pltpu.VMEM((T,), jnp.int32),
          pltpu.VMEM((seg,), jnp.int32),
          pltpu.VMEM((seg,), jnp.int32),
          pltpu.VMEM_SHARED((NS, T), jnp.int32),
      ],
  )
  def k(val_hbm, mi_hbm, di_hbm, out_hbm, val_v, mi_v, di_v, tbl, mseg,
        tseg, spm):
    c = lax.axis_index("c")
    sid = lax.axis_index("s")
    w = sid * NC + c

    def init(i, carry):
      tbl[pl.ds(i * 16, 16)] = jnp.full((16,), 2047, jnp.int32)
      return carry

    lax.fori_loop(0, T // 16, init, 0)
    pltpu.sync_copy(val_hbm.at[w], val_v)
    pltpu.sync_copy(mi_hbm.at[w], mi_v)
    pltpu.sync_copy(di_hbm.at[w], di_v)

    def row(s, carry):
      def vec(kk, carry2):
        m = mi_v[s, pl.ds(kk * 16, 16)]
        d = di_v[s, pl.ds(kk * 16, 16)]
        v = val_v[s, pl.ds(kk * 16, 16)]
        b = plsc.bitcast(v, jnp.int32)
        key32 = jnp.where(b >= 0, b, b ^ jnp.int32(0x7FFFFFFF))
        key = (key32 & jnp.int32(-2048)) | (jnp.int32(2047) - d)

        def w_cond(mask):
          return jnp.max(jnp.where(mask, 1, 0)) > 0

        def w_body(mask):
          plsc.store_scatter(tbl, [m], key, mask=mask)
          cur = plsc.load_gather(tbl, [m])
          return mask & (cur < key)

        lax.while_loop(w_cond, w_body,
                       key > plsc.load_gather(tbl, [m]))
        return carry2

      lax.fori_loop(0, IDXW // 16, vec, carry)
      return carry

    lax.fori_loop(0, steps, row, 0)

    pltpu.sync_copy(tbl, spm.at[sid])
    plsc.subcore_barrier()
    pltpu.sync_copy(spm.at[0, pl.ds(sid * seg, seg)], mseg)

    def merge(j, carry):
      pltpu.sync_copy(spm.at[j, pl.ds(sid * seg, seg)], tseg)

      def mvec(i, carry2):
        mseg[pl.ds(i * 16, 16)] = jnp.maximum(
            mseg[pl.ds(i * 16, 16)], tseg[pl.ds(i * 16, 16)])
        return carry2

      lax.fori_loop(0, seg // 16, mvec, carry)
      return carry

    lax.fori_loop(1, NS, merge, 0)
    pltpu.sync_copy(mseg, out_hbm.at[c, pl.ds(sid * seg, seg)])

  return k(vals3, mi3, di3)


def _tc_mlp(medde, lab, Wa, ba, Wb, bb, Wc, bc, b_real):
  BP = medde.shape[0]
  BLK = 8192
  G = BP // BLK
  s2 = Wc.shape[0]
  s_dim = s2 // 2

  def body(x_ref, lab_ref, Wa_r, ba_r, Wb_r, bb_r, Wc_r, bc_r,
           pay_ref, msds_ref, bce_ref):
    i = pl.program_id(0)

    def dnT(w_ref, xT, dims):
      return lax.dot_general(w_ref[...], xT, (dims, ((), ())),
                             preferred_element_type=jnp.float32)

    pmdT = dnT(Wa_r, x_ref[...], ((1,), (1,))) + ba_r[...]
    hT = jnp.maximum(dnT(Wb_r, pmdT, ((1,), (0,))) + bb_r[...], 0.0)
    msdsT = dnT(Wc_r, hT, ((1,), (0,))) + bc_r[...]
    msT = msdsT[:s_dim, :]
    dsT = msdsT[s_dim:, :]
    num = jnp.sum(msT * dsT, axis=0, keepdims=True)
    den = jnp.sqrt(jnp.sum(msT * msT, axis=0, keepdims=True)) * \
        jnp.sqrt(jnp.sum(dsT * dsT, axis=0, keepdims=True))
    x = num / den
    pay_ref[...] = x[None]
    msds_ref[...] = msdsT.T
    y = lab_ref[0]
    bce = jnp.maximum(x, 0.0) - x * y + jnp.log1p(jnp.exp(-jnp.abs(x)))
    cidx = i * BLK + lax.broadcasted_iota(jnp.int32, (1, BLK), 1)
    s = jnp.sum(jnp.where(cidx < b_real, bce, 0.0), axis=(0, 1),
                keepdims=True)
    bce_ref[...] = jnp.where(i == 0, s, bce_ref[...] + s)

  rep = lambda shape: pl.BlockSpec(shape, lambda i: tuple(0 for _ in shape))
  return pl.pallas_call(
      body,
      grid=(G,),
      in_specs=[
          pl.BlockSpec((BLK, medde.shape[1]), lambda i: (i, 0)),
          pl.BlockSpec((1, 1, BLK), lambda i: (i, 0, 0)),
          rep(Wa.shape), rep(ba.shape), rep(Wb.shape), rep(bb.shape),
          rep(Wc.shape), rep(bc.shape),
      ],
      out_specs=[
          pl.BlockSpec((1, 1, BLK), lambda i: (i, 0, 0)),
          pl.BlockSpec((BLK, s2), lambda i: (i, 0)),
          pl.BlockSpec((1, 1), lambda i: (0, 0)),
      ],
      out_shape=[
          jax.ShapeDtypeStruct((G, 1, BLK), jnp.float32),
          jax.ShapeDtypeStruct((BP, s2), jnp.float32),
          jax.ShapeDtypeStruct((1, 1), jnp.float32),
      ],
  )(medde, lab, Wa, ba, Wb, bb, Wc, bc)


def _tc_unpack(acc, T):
  def body(a_ref, o_ref):
    mx = jnp.max(a_ref[...], axis=0, keepdims=True)
    o_ref[...] = jnp.int32(2047) - (mx & jnp.int32(2047))

  return pl.pallas_call(
      body,
      out_shape=jax.ShapeDtypeStruct((1, T), jnp.int32),
  )(acc)


def _tc_final(msds, big, bce, b_real, s_dim):
  BP = msds.shape[0]
  BLK = 4096
  G = BP // BLK

  def body(a_ref, g_ref, bce_ref, out_ref):
    i = pl.program_id(0)
    d = a_ref[...] - g_ref[...]
    ridx = i * BLK + lax.broadcasted_iota(jnp.int32, (BLK, 1), 0)
    s = jnp.sum(jnp.where(ridx < b_real, d * d, 0.0), axis=(0, 1),
                keepdims=True)
    acc = jnp.where(i == 0, s, out_ref[...] + s)
    out_ref[...] = jnp.where(
        i == G - 1,
        acc / (2.0 * b_real * s_dim) + bce_ref[...] / b_real,
        acc)

  return pl.pallas_call(
      body,
      grid=(G,),
      in_specs=[
          pl.BlockSpec((BLK, 2 * s_dim), lambda i: (i, 0)),
          pl.BlockSpec((BLK, 2 * s_dim), lambda i: (i, 0)),
          pl.BlockSpec((1, 1), lambda i: (0, 0)),
      ],
      out_specs=pl.BlockSpec((1, 1), lambda i: (0, 0)),
      out_shape=jax.ShapeDtypeStruct((1, 1), jnp.float32),
  )(msds, big, bce)


def kernel(miRNA_embeddings, disease_embeddings, miRNA_index, disease_index,
           true_labels, W_m, b_m, W_d, b_d, W_ms1, b_ms1, W_ms2, b_ms2,
           W_ds1, b_ds1, W_ds2, b_ds2):
  num_m = miRNA_embeddings.shape[0]
  num_d = disease_embeddings.shape[0]
  B = miRNA_index.shape[0]
  s_dim = W_ms2.shape[0]

  BP = 16384 * (-(-B // 16384))
  steps = BP // (NW * IDXW)
  pad = BP - B
  mi = miRNA_index.astype(jnp.int32)
  di = disease_index.astype(jnp.int32)
  mi_g = jnp.pad(mi, (0, pad))
  di_g = jnp.pad(di, (0, pad))
  mi_s = jnp.pad(mi, (0, pad), constant_values=num_m)
  lab = jnp.pad(true_labels, (0, pad)).reshape(BP // 8192, 1, 8192)

  mi2 = mi_g.reshape(NW, steps, IDXW)
  di2 = di_g.reshape(NW, steps, IDXW)

  medde = _sc_gather2(miRNA_embeddings, disease_embeddings, mi2, di2, steps)

  h = W_m.shape[0]
  fm = W_m.shape[1]
  fd = W_d.shape[1]
  s_dim = W_ms2.shape[0]
  Wa = jnp.zeros((2 * h, fm + fd), jnp.float32)
  Wa = Wa.at[:h, :fm].set(W_m).at[h:, fm:].set(W_d)
  ba = jnp.concatenate([b_m, b_d]).reshape(2 * h, 1)
  Wb = jnp.zeros((2 * h, 2 * h), jnp.float32)
  Wb = Wb.at[:h, :h].set(W_ms1).at[h:, h:].set(W_ds1)
  bb = jnp.concatenate([b_ms1, b_ds1]).reshape(2 * h, 1)
  Wc = jnp.zeros((2 * s_dim, 2 * h), jnp.float32)
  Wc = Wc.at[:s_dim, :h].set(W_ms2).at[s_dim:, h:].set(W_ds2)
  bc = jnp.concatenate([b_ms2, b_ds2]).reshape(2 * s_dim, 1)

  pay, msds, bce = _tc_mlp(medde, lab, Wa, ba, Wb, bb, Wc, bc, B)

  bi_n = NW * IDXW * (-(-num_m // (NW * IDXW)))
  bi_steps = bi_n // (NW * IDXW)
  acc = _sc_segmax(pay.reshape(NW, steps, IDXW),
                   mi_s.reshape(NW, steps, IDXW), di2, steps, bi_n)
  bi2 = _tc_unpack(acc, bi_n)

  bi_pad = bi2.reshape(NW, bi_steps, IDXW)
  g2 = _sc_gather(msds, bi_pad, bi_steps)
  big = _sc_gatherp(g2, mi2, steps)

  loss = _tc_final(msds, big, bce, B, s_dim)
  return (pay.reshape(-1)[:B], loss[0, 0])


# speedup vs baseline: 8.7525x; 1.0724x over previous
import functools

import jax
import jax.numpy as jnp
from jax import lax
from jax.experimental import pallas as pl
from jax.experimental.pallas import tpu as pltpu
from jax.experimental.pallas import tpu_sc as plsc

NC, NS = 2, 16
NW = NC * NS
IDXW = 128


def _sc_mesh():
  return plsc.VectorSubcoreMesh(
      core_axis_name="c", subcore_axis_name="s",
      num_cores=NC, num_subcores=NS)


def _sc_gather(table, idx3, steps):
  _, D = table.shape
  N = NW * steps * IDXW

  @functools.partial(
      pl.kernel,
      out_type=jax.ShapeDtypeStruct((N, D), jnp.float32),
      mesh=_sc_mesh(),
      compiler_params=pltpu.CompilerParams(use_tc_tiling_on_sc=False),
      scratch_types=[
          pltpu.VMEM((steps, IDXW), jnp.int32),
          pltpu.VMEM((IDXW, D), jnp.float32),
          pltpu.SemaphoreType.DMA,
      ],
  )
  def k(table_hbm, idx_hbm, out_hbm, idx_v, rows_v, sem):
    w = lax.axis_index("s") * NC + lax.axis_index("c")
    pltpu.sync_copy(idx_hbm.at[w], idx_v)

    def body(s, carry):
      pltpu.async_copy(table_hbm.at[idx_v.at[s]], rows_v, sem).wait()
      pltpu.sync_copy(rows_v, out_hbm.at[pl.ds((w * steps + s) * IDXW, IDXW)])
      return carry

    lax.fori_loop(0, steps, body, 0)

  return k(table, idx3)


def _sc_gatherp(table, idx3, steps):
  _, D = table.shape
  N = NW * steps * IDXW

  @functools.partial(
      pl.kernel,
      out_type=jax.ShapeDtypeStruct((N, D), jnp.float32),
      mesh=_sc_mesh(),
      compiler_params=pltpu.CompilerParams(use_tc_tiling_on_sc=False),
      scratch_types=[
          pltpu.VMEM((steps, IDXW), jnp.int32),
          pltpu.VMEM((IDXW, D), jnp.float32),
          pltpu.VMEM((IDXW, D), jnp.float32),
          pltpu.SemaphoreType.DMA,
          pltpu.SemaphoreType.DMA,
      ],
  )
  def k(t_hbm, idx_hbm, out_hbm, idx_v, r0, r1, s0, s1):
    w = lax.axis_index("s") * NC + lax.axis_index("c")
    base = w * steps
    pltpu.sync_copy(idx_hbm.at[w], idx_v)
    pltpu.async_copy(t_hbm.at[idx_v.at[0]], r0, s0)

    def body(t, carry):
      sa = 2 * t
      sb = 2 * t + 1
      pltpu.async_copy(t_hbm.at[idx_v.at[sb]], r1, s1)
      pltpu.make_async_copy(t_hbm.at[idx_v.at[sa]], r0, s0).wait()
      pltpu.sync_copy(r0, out_hbm.at[pl.ds((base + sa) * IDXW, IDXW)])

      @pl.when(sa + 2 < steps)
      def _():
        pltpu.async_copy(t_hbm.at[idx_v.at[sa + 2]], r0, s0)

      pltpu.make_async_copy(t_hbm.at[idx_v.at[sb]], r1, s1).wait()
      pltpu.sync_copy(r1, out_hbm.at[pl.ds((base + sb) * IDXW, IDXW)])
      return carry

    lax.fori_loop(0, steps // 2, body, 0)

  return k(table, idx3)


def _sc_gather2(tab_a, tab_b, idx_a3, idx_b3, steps):
  Da = tab_a.shape[1]
  Db = tab_b.shape[1]
  N = NW * steps * IDXW

  @functools.partial(
      pl.kernel,
      out_type=jax.ShapeDtypeStruct((N, Da + Db), jnp.float32),
      mesh=_sc_mesh(),
      compiler_params=pltpu.CompilerParams(use_tc_tiling_on_sc=False),
      scratch_types=[
          pltpu.VMEM((steps, IDXW), jnp.int32),
          pltpu.VMEM((steps, IDXW), jnp.int32),
          pltpu.VMEM((IDXW, Da), jnp.float32),
          pltpu.VMEM((IDXW, Da), jnp.float32),
          pltpu.VMEM((IDXW, Db), jnp.float32),
          pltpu.VMEM((IDXW, Db), jnp.float32),
          pltpu.SemaphoreType.DMA,
          pltpu.SemaphoreType.DMA,
          pltpu.SemaphoreType.DMA,
          pltpu.SemaphoreType.DMA,
      ],
  )
  def k(ta_hbm, tb_hbm, ia_hbm, ib_hbm, out_hbm, ia_v, ib_v, ra0, ra1,
        rb0, rb1, sa0, sa1, sb0, sb1):
    w = lax.axis_index("s") * NC + lax.axis_index("c")
    base = w * steps
    pltpu.sync_copy(ia_hbm.at[w], ia_v)
    pltpu.sync_copy(ib_hbm.at[w], ib_v)
    pltpu.async_copy(ta_hbm.at[ia_v.at[0]], ra0, sa0)
    pltpu.async_copy(tb_hbm.at[ib_v.at[0]], rb0, sb0)

    def body(t, carry):
      sa = 2 * t
      sb = 2 * t + 1
      pltpu.async_copy(ta_hbm.at[ia_v.at[sb]], ra1, sa1)
      pltpu.async_copy(tb_hbm.at[ib_v.at[sb]], rb1, sb1)
      r0 = (base + sa) * IDXW
      pltpu.make_async_copy(ta_hbm.at[ia_v.at[sa]], ra0, sa0).wait()
      pltpu.sync_copy(ra0, out_hbm.at[pl.ds(r0, IDXW), pl.ds(0, Da)])
      pltpu.make_async_copy(tb_hbm.at[ib_v.at[sa]], rb0, sb0).wait()
      pltpu.sync_copy(rb0, out_hbm.at[pl.ds(r0, IDXW), pl.ds(Da, Db)])

      @pl.when(sa + 2 < steps)
      def _():
        pltpu.async_copy(ta_hbm.at[ia_v.at[sa + 2]], ra0, sa0)
        pltpu.async_copy(tb_hbm.at[ib_v.at[sa + 2]], rb0, sb0)

      r1 = (base + sb) * IDXW
      pltpu.make_async_copy(ta_hbm.at[ia_v.at[sb]], ra1, sa1).wait()
      pltpu.sync_copy(ra1, out_hbm.at[pl.ds(r1, IDXW), pl.ds(0, Da)])
      pltpu.make_async_copy(tb_hbm.at[ib_v.at[sb]], rb1, sb1).wait()
      pltpu.sync_copy(rb1, out_hbm.at[pl.ds(r1, IDXW), pl.ds(Da, Db)])
      return carry

    lax.fori_loop(0, steps // 2, body, 0)

  return k(tab_a, tab_b, idx_a3, idx_b3)


def _sc_segmax(vals3, mi3, di3, steps, T):
  seg = T // NS

  @functools.partial(
      pl.kernel,
      out_type=jax.ShapeDtypeStruct((NC, T), jnp.int32),
      mesh=_sc_mesh(),
      compiler_params=pltpu.CompilerParams(use_tc_tiling_on_sc=False,
                                           needs_layout_passes=False),
      scratch_types=[
          pltpu.VMEM((steps, IDXW), jnp.float32),
          pltpu.VMEM((steps, IDXW), jnp.int32),
          pltpu.VMEM((steps, IDXW), jnp.int32),
          pltpu.VMEM((T,), jnp.int32),
          pltpu.VMEM((seg,), jnp.int32),
          pltpu.VMEM((seg,), jnp.int32),
          pltpu.VMEM_SHARED((NS, T), jnp.int32),
      ],
  )
  def k(val_hbm, mi_hbm, di_hbm, out_hbm, val_v, mi_v, di_v, tbl, mseg,
        tseg, spm):
    c = lax.axis_index("c")
    sid = lax.axis_index("s")
    w = sid * NC + c

    def init(i, carry):
      tbl[pl.ds(i * 16, 16)] = jnp.full((16,), 2047, jnp.int32)
      return carry

    lax.fori_loop(0, T // 16, init, 0)
    pltpu.sync_copy(val_hbm.at[w], val_v)
    pltpu.sync_copy(mi_hbm.at[w], mi_v)
    pltpu.sync_copy(di_hbm.at[w], di_v)

    def row(s, carry):
      def vec(kk, carry2):
        m = mi_v[s, pl.ds(kk * 16, 16)]
        d = di_v[s, pl.ds(kk * 16, 16)]
        v = val_v[s, pl.ds(kk * 16, 16)]
        b = plsc.bitcast(v, jnp.int32)
        key32 = jnp.where(b >= 0, b, b ^ jnp.int32(0x7FFFFFFF))
        key = (key32 & jnp.int32(-2048)) | (jnp.int32(2047) - d)

        def w_cond(mask):
          return jnp.max(jnp.where(mask, 1, 0)) > 0

        def w_body(mask):
          plsc.store_scatter(tbl, [m], key, mask=mask)
          cur = plsc.load_gather(tbl, [m])
          return mask & (cur < key)

        lax.while_loop(w_cond, w_body,
                       key > plsc.load_gather(tbl, [m]))
        return carry2

      lax.fori_loop(0, IDXW // 16, vec, carry)
      return carry

    lax.fori_loop(0, steps, row, 0)

    pltpu.sync_copy(tbl, spm.at[sid])
    plsc.subcore_barrier()
    pltpu.sync_copy(spm.at[0, pl.ds(sid * seg, seg)], mseg)

    def merge(j, carry):
      pltpu.sync_copy(spm.at[j, pl.ds(sid * seg, seg)], tseg)

      def mvec(i, carry2):
        mseg[pl.ds(i * 16, 16)] = jnp.maximum(
            mseg[pl.ds(i * 16, 16)], tseg[pl.ds(i * 16, 16)])
        return carry2

      lax.fori_loop(0, seg // 16, mvec, carry)
      return carry

    lax.fori_loop(1, NS, merge, 0)
    pltpu.sync_copy(mseg, out_hbm.at[c, pl.ds(sid * seg, seg)])

  return k(vals3, mi3, di3)


def _sc_gsel(wideT, bi3, steps, s2, num_d):
  N = NW * steps * IDXW
  RD = -(-num_d // IDXW) * IDXW

  @functools.partial(
      pl.kernel,
      out_type=jax.ShapeDtypeStruct((N, s2), jnp.float32),
      mesh=_sc_mesh(),
      compiler_params=pltpu.CompilerParams(use_tc_tiling_on_sc=False,
                                           needs_layout_passes=False),
      scratch_types=[
          pltpu.VMEM((steps, IDXW), jnp.int32),
          pltpu.VMEM((32, 2048), jnp.float32),
          pltpu.VMEM((IDXW, 32), jnp.float32),
      ],
  )
  def k(w_hbm, bi_hbm, out_hbm, bi_v, reg, sel_v):
    w = lax.axis_index("s") * NC + lax.axis_index("c")
    pltpu.sync_copy(bi_hbm.at[w], bi_v)
    pltpu.sync_copy(w_hbm.at[pl.ds(0, s2), pl.ds(0, RD)],
                    reg.at[pl.ds(0, s2), pl.ds(0, RD)])

    def step(s, carry):
      def jc(q, c2):
        rvec = lax.iota(jnp.int32, 16) + q * 16
        bivec = bi_v[s, pl.ds(q * 16, 16)]

        def tcol(t, c3):
          tv = jnp.full((16,), 0, jnp.int32) + t
          vals = plsc.load_gather(reg, [tv, bivec])
          plsc.store_scatter(sel_v, [rvec, tv], vals)
          return c3

        lax.fori_loop(0, s2, tcol, 0)
        return c2

      lax.fori_loop(0, IDXW // 16, jc, 0)
      pltpu.sync_copy(sel_v.at[pl.ds(0, IDXW), pl.ds(0, s2)],
                      out_hbm.at[pl.ds((w * steps + s) * IDXW, IDXW)])
      return carry

    lax.fori_loop(0, steps, step, 0)

  return k(wideT, bi3)


def _sc_nash(g2, wideT, mi3, steps, s2, blk):
  @functools.partial(
      pl.kernel,
      out_type=jax.ShapeDtypeStruct((NW, 16), jnp.float32),
      mesh=_sc_mesh(),
      compiler_params=pltpu.CompilerParams(use_tc_tiling_on_sc=False,
                                           needs_layout_passes=False),
      scratch_types=[
          pltpu.VMEM((steps, IDXW), jnp.int32),
          pltpu.VMEM((IDXW, 32), jnp.float32),
          pltpu.VMEM((32, IDXW), jnp.float32),
          pltpu.VMEM((16,), jnp.float32),
          pltpu.SemaphoreType.DMA,
      ],
  )
  def k(g2_hbm, w_hbm, mi_hbm, out_hbm, mi_v, gbuf, mbuf, acc, sem):
    w = lax.axis_index("s") * NC + lax.axis_index("c")
    pltpu.sync_copy(mi_hbm.at[w], mi_v)
    acc[...] = jnp.zeros((16,), jnp.float32)

    def step(s, carry):
      p0 = (w * steps + s) * IDXW
      bk = p0 // blk
      c0 = p0 - bk * blk
      pltpu.async_copy(g2_hbm.at[mi_v.at[s]],
                       gbuf.at[pl.ds(0, IDXW), pl.ds(0, s2)], sem).wait()
      pltpu.sync_copy(w_hbm.at[pl.ds(bk * s2, s2), pl.ds(c0, IDXW)],
                      mbuf.at[pl.ds(0, s2), pl.ds(0, IDXW)])

      def ft(j, c2):
        jv = jnp.full((16,), 0, jnp.int32) + j
        t = acc[...]
        for q in range(IDXW // 16):
          a = mbuf[j, pl.ds(q * 16, 16)]
          rvec = lax.iota(jnp.int32, 16) + q * 16
          b2 = plsc.load_gather(gbuf, [rvec, jv])
          dd = a - b2
          t = t + dd * dd
        acc[...] = t
        return c2

      lax.fori_loop(0, s2, ft, 0)
      return carry

    lax.fori_loop(0, steps, step, 0)
    pltpu.sync_copy(acc, out_hbm.at[w])

  return k(g2, wideT, mi3)


def _tc_mlp(medde, lab, Wa, ba, Wb, bb, Wc, bc, b_real):
  BP = medde.shape[0]
  BLK = 8192
  G = BP // BLK
  s2 = Wc.shape[0]
  s_dim = s2 // 2

  def body(x_ref, lab_ref, Wa_r, ba_r, Wb_r, bb_r, Wc_r, bc_r,
           pay_ref, msds_ref, bce_ref):
    i = pl.program_id(0)

    def dnT(w_ref, xT, dims):
      return lax.dot_general(w_ref[...], xT, (dims, ((), ())),
                             preferred_element_type=jnp.float32)

    pmdT = dnT(Wa_r, x_ref[...], ((1,), (1,))) + ba_r[...]
    hT = jnp.maximum(dnT(Wb_r, pmdT, ((1,), (0,))) + bb_r[...], 0.0)
    msdsT = dnT(Wc_r, hT, ((1,), (0,))) + bc_r[...]
    msT = msdsT[:s_dim, :]
    dsT = msdsT[s_dim:, :]
    num = jnp.sum(msT * dsT, axis=0, keepdims=True)
    den = jnp.sqrt(jnp.sum(msT * msT, axis=0, keepdims=True)) * \
        jnp.sqrt(jnp.sum(dsT * dsT, axis=0, keepdims=True))
    x = num / den
    pay_ref[...] = x[None]
    msds_ref[...] = msdsT
    y = lab_ref[0]
    bce = jnp.maximum(x, 0.0) - x * y + jnp.log1p(jnp.exp(-jnp.abs(x)))
    cidx = i * BLK + lax.broadcasted_iota(jnp.int32, (1, BLK), 1)
    s = jnp.sum(jnp.where(cidx < b_real, bce, 0.0), axis=(0, 1),
                keepdims=True)
    bce_ref[...] = jnp.where(i == 0, s, bce_ref[...] + s)

  rep = lambda shape: pl.BlockSpec(shape, lambda i: tuple(0 for _ in shape))
  return pl.pallas_call(
      body,
      grid=(G,),
      in_specs=[
          pl.BlockSpec((BLK, medde.shape[1]), lambda i: (i, 0)),
          pl.BlockSpec((1, 1, BLK), lambda i: (i, 0, 0)),
          rep(Wa.shape), rep(ba.shape), rep(Wb.shape), rep(bb.shape),
          rep(Wc.shape), rep(bc.shape),
      ],
      out_specs=[
          pl.BlockSpec((1, 1, BLK), lambda i: (i, 0, 0)),
          pl.BlockSpec((s2, BLK), lambda i: (i, 0)),
          pl.BlockSpec((1, 1), lambda i: (0, 0)),
      ],
      out_shape=[
          jax.ShapeDtypeStruct((G, 1, BLK), jnp.float32),
          jax.ShapeDtypeStruct((G * s2, BLK), jnp.float32),
          jax.ShapeDtypeStruct((1, 1), jnp.float32),
      ],
  )(medde, lab, Wa, ba, Wb, bb, Wc, bc)


def _tc_unpack(acc, T):
  def body(a_ref, o_ref):
    mx = jnp.max(a_ref[...], axis=0, keepdims=True)
    o_ref[...] = jnp.int32(2047) - (mx & jnp.int32(2047))

  return pl.pallas_call(
      body,
      out_shape=jax.ShapeDtypeStruct((1, T), jnp.int32),
  )(acc)


def _tc_loss(partials, padblk, g2row, bce, b_real, s_dim):
  def body(p_ref, pb_ref, gr_ref, bce_ref, out_ref):
    tot = jnp.sum(p_ref[...], axis=(0, 1), keepdims=True)
    dd = pb_ref[...] - gr_ref[...]
    pad_sum = jnp.sum(dd * dd, axis=(0, 1), keepdims=True)
    out_ref[...] = (tot - pad_sum) / (2.0 * b_real * s_dim) + \
        bce_ref[...] / b_real

  return pl.pallas_call(
      body,
      out_shape=jax.ShapeDtypeStruct((1, 1), jnp.float32),
  )(partials, padblk, g2row, bce)


def kernel(miRNA_embeddings, disease_embeddings, miRNA_index, disease_index,
           true_labels, W_m, b_m, W_d, b_d, W_ms1, b_ms1, W_ms2, b_ms2,
           W_ds1, b_ds1, W_ds2, b_ds2):
  num_m = miRNA_embeddings.shape[0]
  num_d = disease_embeddings.shape[0]
  B = miRNA_index.shape[0]
  s_dim = W_ms2.shape[0]

  BP = 16384 * (-(-B // 16384))
  steps = BP // (NW * IDXW)
  pad = BP - B
  mi = miRNA_index.astype(jnp.int32)
  di = disease_index.astype(jnp.int32)
  mi_g = jnp.pad(mi, (0, pad))
  di_g = jnp.pad(di, (0, pad))
  mi_s = jnp.pad(mi, (0, pad), constant_values=num_m)
  lab = jnp.pad(true_labels, (0, pad)).reshape(BP // 8192, 1, 8192)

  mi2 = mi_g.reshape(NW, steps, IDXW)
  di2 = di_g.reshape(NW, steps, IDXW)

  medde = _sc_gather2(miRNA_embeddings, disease_embeddings, mi2, di2, steps)

  h = W_m.shape[0]
  fm = W_m.shape[1]
  fd = W_d.shape[1]
  s_dim = W_ms2.shape[0]
  Wa = jnp.zeros((2 * h, fm + fd), jnp.float32)
  Wa = Wa.at[:h, :fm].set(W_m).at[h:, fm:].set(W_d)
  ba = jnp.concatenate([b_m, b_d]).reshape(2 * h, 1)
  Wb = jnp.zeros((2 * h, 2 * h), jnp.float32)
  Wb = Wb.at[:h, :h].set(W_ms1).at[h:, h:].set(W_ds1)
  bb = jnp.concatenate([b_ms1, b_ds1]).reshape(2 * h, 1)
  Wc = jnp.zeros((2 * s_dim, 2 * h), jnp.float32)
  Wc = Wc.at[:s_dim, :h].set(W_ms2).at[s_dim:, h:].set(W_ds2)
  bc = jnp.concatenate([b_ms2, b_ds2]).reshape(2 * s_dim, 1)

  pay, msds, bce = _tc_mlp(medde, lab, Wa, ba, Wb, bb, Wc, bc, B)

  bi_n = NW * IDXW * (-(-num_m // (NW * IDXW)))
  bi_steps = bi_n // (NW * IDXW)
  acc = _sc_segmax(pay.reshape(NW, steps, IDXW),
                   mi_s.reshape(NW, steps, IDXW), di2, steps, bi_n)
  bi2 = _tc_unpack(acc, bi_n)

  s2 = 2 * s_dim
  bi_pad = bi2.reshape(NW, bi_steps, IDXW)
  g2 = _sc_gsel(msds, bi_pad, bi_steps, s2, num_d)
  partials = _sc_nash(g2, msds, mi2, steps, s2, 8192)

  bk = B // 8192
  padT = msds[bk * s2:(bk + 1) * s2, B - bk * 8192:]
  g2col = jnp.transpose(g2[0:1, :])
  loss = _tc_loss(partials, padT, g2col, bce, B, s_dim)
  return (pay.reshape(-1)[:B], loss[0, 0])


# speedup vs baseline: 10.5427x; 1.2045x over previous
import functools

import jax
import jax.numpy as jnp
from jax import lax
from jax.experimental import pallas as pl
from jax.experimental.pallas import tpu as pltpu
from jax.experimental.pallas import tpu_sc as plsc

NC, NS = 2, 16
NW = NC * NS
IDXW = 128


def _sc_mesh():
  return plsc.VectorSubcoreMesh(
      core_axis_name="c", subcore_axis_name="s",
      num_cores=NC, num_subcores=NS)


def _sc_gather(table, idx3, steps):
  _, D = table.shape
  N = NW * steps * IDXW

  @functools.partial(
      pl.kernel,
      out_type=jax.ShapeDtypeStruct((N, D), jnp.float32),
      mesh=_sc_mesh(),
      compiler_params=pltpu.CompilerParams(use_tc_tiling_on_sc=False),
      scratch_types=[
          pltpu.VMEM((steps, IDXW), jnp.int32),
          pltpu.VMEM((IDXW, D), jnp.float32),
          pltpu.SemaphoreType.DMA,
      ],
  )
  def k(table_hbm, idx_hbm, out_hbm, idx_v, rows_v, sem):
    w = lax.axis_index("s") * NC + lax.axis_index("c")
    pltpu.sync_copy(idx_hbm.at[w], idx_v)

    def body(s, carry):
      pltpu.async_copy(table_hbm.at[idx_v.at[s]], rows_v, sem).wait()
      pltpu.sync_copy(rows_v, out_hbm.at[pl.ds((w * steps + s) * IDXW, IDXW)])
      return carry

    lax.fori_loop(0, steps, body, 0)

  return k(table, idx3)


def _sc_gatherp(table, idx3, steps):
  _, D = table.shape
  N = NW * steps * IDXW

  @functools.partial(
      pl.kernel,
      out_type=jax.ShapeDtypeStruct((N, D), jnp.float32),
      mesh=_sc_mesh(),
      compiler_params=pltpu.CompilerParams(use_tc_tiling_on_sc=False),
      scratch_types=[
          pltpu.VMEM((steps, IDXW), jnp.int32),
          pltpu.VMEM((IDXW, D), jnp.float32),
          pltpu.VMEM((IDXW, D), jnp.float32),
          pltpu.SemaphoreType.DMA,
          pltpu.SemaphoreType.DMA,
      ],
  )
  def k(t_hbm, idx_hbm, out_hbm, idx_v, r0, r1, s0, s1):
    w = lax.axis_index("s") * NC + lax.axis_index("c")
    base = w * steps
    pltpu.sync_copy(idx_hbm.at[w], idx_v)
    pltpu.async_copy(t_hbm.at[idx_v.at[0]], r0, s0)

    def body(t, carry):
      sa = 2 * t
      sb = 2 * t + 1
      pltpu.async_copy(t_hbm.at[idx_v.at[sb]], r1, s1)
      pltpu.make_async_copy(t_hbm.at[idx_v.at[sa]], r0, s0).wait()
      pltpu.sync_copy(r0, out_hbm.at[pl.ds((base + sa) * IDXW, IDXW)])

      @pl.when(sa + 2 < steps)
      def _():
        pltpu.async_copy(t_hbm.at[idx_v.at[sa + 2]], r0, s0)

      pltpu.make_async_copy(t_hbm.at[idx_v.at[sb]], r1, s1).wait()
      pltpu.sync_copy(r1, out_hbm.at[pl.ds((base + sb) * IDXW, IDXW)])
      return carry

    lax.fori_loop(0, steps // 2, body, 0)

  return k(table, idx3)


def _sc_gather2(tab_a, tab_b, idx_a3, idx_b3, steps):
  Da = tab_a.shape[1]
  Db = tab_b.shape[1]
  N = NW * steps * IDXW

  @functools.partial(
      pl.kernel,
      out_type=jax.ShapeDtypeStruct((N, Da + Db), jnp.float32),
      mesh=_sc_mesh(),
      compiler_params=pltpu.CompilerParams(use_tc_tiling_on_sc=False),
      scratch_types=[
          pltpu.VMEM((steps, IDXW), jnp.int32),
          pltpu.VMEM((steps, IDXW), jnp.int32),
          pltpu.VMEM((IDXW, Da), jnp.float32),
          pltpu.VMEM((IDXW, Da), jnp.float32),
          pltpu.VMEM((IDXW, Db), jnp.float32),
          pltpu.VMEM((IDXW, Db), jnp.float32),
          pltpu.SemaphoreType.DMA,
          pltpu.SemaphoreType.DMA,
          pltpu.SemaphoreType.DMA,
          pltpu.SemaphoreType.DMA,
      ],
  )
  def k(ta_hbm, tb_hbm, ia_hbm, ib_hbm, out_hbm, ia_v, ib_v, ra0, ra1,
        rb0, rb1, sa0, sa1, sb0, sb1):
    w = lax.axis_index("s") * NC + lax.axis_index("c")
    base = w * steps
    pltpu.sync_copy(ia_hbm.at[w], ia_v)
    pltpu.sync_copy(ib_hbm.at[w], ib_v)
    pltpu.async_copy(ta_hbm.at[ia_v.at[0]], ra0, sa0)
    pltpu.async_copy(tb_hbm.at[ib_v.at[0]], rb0, sb0)

    def body(t, carry):
      sa = 2 * t
      sb = 2 * t + 1
      pltpu.async_copy(ta_hbm.at[ia_v.at[sb]], ra1, sa1)
      pltpu.async_copy(tb_hbm.at[ib_v.at[sb]], rb1, sb1)
      r0 = (base + sa) * IDXW
      pltpu.make_async_copy(ta_hbm.at[ia_v.at[sa]], ra0, sa0).wait()
      pltpu.sync_copy(ra0, out_hbm.at[pl.ds(r0, IDXW), pl.ds(0, Da)])
      pltpu.make_async_copy(tb_hbm.at[ib_v.at[sa]], rb0, sb0).wait()
      pltpu.sync_copy(rb0, out_hbm.at[pl.ds(r0, IDXW), pl.ds(Da, Db)])

      @pl.when(sa + 2 < steps)
      def _():
        pltpu.async_copy(ta_hbm.at[ia_v.at[sa + 2]], ra0, sa0)
        pltpu.async_copy(tb_hbm.at[ib_v.at[sa + 2]], rb0, sb0)

      r1 = (base + sb) * IDXW
      pltpu.make_async_copy(ta_hbm.at[ia_v.at[sb]], ra1, sa1).wait()
      pltpu.sync_copy(ra1, out_hbm.at[pl.ds(r1, IDXW), pl.ds(0, Da)])
      pltpu.make_async_copy(tb_hbm.at[ib_v.at[sb]], rb1, sb1).wait()
      pltpu.sync_copy(rb1, out_hbm.at[pl.ds(r1, IDXW), pl.ds(Da, Db)])
      return carry

    lax.fori_loop(0, steps // 2, body, 0)

  return k(tab_a, tab_b, idx_a3, idx_b3)


def _sc_segmax(vals3, mi3, di3, steps, T):
  seg = T // NS

  @functools.partial(
      pl.kernel,
      out_type=jax.ShapeDtypeStruct((NC, T), jnp.int32),
      mesh=_sc_mesh(),
      compiler_params=pltpu.CompilerParams(use_tc_tiling_on_sc=False,
                                           needs_layout_passes=False),
      scratch_types=[
          pltpu.VMEM((steps, IDXW), jnp.float32),
          pltpu.VMEM((steps, IDXW), jnp.int32),
          pltpu.VMEM((steps, IDXW), jnp.int32),
          pltpu.VMEM((T,), jnp.int32),
          pltpu.VMEM((seg,), jnp.int32),
          pltpu.VMEM((seg,), jnp.int32),
          pltpu.VMEM_SHARED((NS, T), jnp.int32),
      ],
  )
  def k(val_hbm, mi_hbm, di_hbm, out_hbm, val_v, mi_v, di_v, tbl, mseg,
        tseg, spm):
    c = lax.axis_index("c")
    sid = lax.axis_index("s")
    w = sid * NC + c

    def init(i, carry):
      tbl[pl.ds(i * 16, 16)] = jnp.full((16,), 2047, jnp.int32)
      return carry

    lax.fori_loop(0, T // 16, init, 0)
    pltpu.sync_copy(val_hbm.at[w], val_v)
    pltpu.sync_copy(mi_hbm.at[w], mi_v)
    pltpu.sync_copy(di_hbm.at[w], di_v)

    def row(s, carry):
      def vec(kk, carry2):
        m = mi_v[s, pl.ds(kk * 16, 16)]
        d = di_v[s, pl.ds(kk * 16, 16)]
        v = val_v[s, pl.ds(kk * 16, 16)]
        b = plsc.bitcast(v, jnp.int32)
        key32 = jnp.where(b >= 0, b, b ^ jnp.int32(0x7FFFFFFF))
        key = (key32 & jnp.int32(-2048)) | (jnp.int32(2047) - d)

        def w_cond(mask):
          return jnp.max(jnp.where(mask, 1, 0)) > 0

        def w_body(mask):
          plsc.store_scatter(tbl, [m], key, mask=mask)
          cur = plsc.load_gather(tbl, [m])
          return mask & (cur < key)

        lax.while_loop(w_cond, w_body,
                       key > plsc.load_gather(tbl, [m]))
        return carry2

      lax.fori_loop(0, IDXW // 16, vec, carry)
      return carry

    lax.fori_loop(0, steps, row, 0)

    pltpu.sync_copy(tbl, spm.at[sid])
    plsc.subcore_barrier()
    pltpu.sync_copy(spm.at[0, pl.ds(sid * seg, seg)], mseg)

    def merge(j, carry):
      pltpu.sync_copy(spm.at[j, pl.ds(sid * seg, seg)], tseg)

      def mvec(i, carry2):
        mseg[pl.ds(i * 16, 16)] = jnp.maximum(
            mseg[pl.ds(i * 16, 16)], tseg[pl.ds(i * 16, 16)])
        return carry2

      lax.fori_loop(0, seg // 16, mvec, carry)
      return carry

    lax.fori_loop(1, NS, merge, 0)
    pltpu.sync_copy(mseg, out_hbm.at[c, pl.ds(sid * seg, seg)])

  return k(vals3, mi3, di3)


def _sc_gsel(wideT, bi3, steps, s2, num_d):
  N = NW * steps * IDXW
  RD = -(-num_d // IDXW) * IDXW

  @functools.partial(
      pl.kernel,
      out_type=jax.ShapeDtypeStruct((N, s2), jnp.float32),
      mesh=_sc_mesh(),
      compiler_params=pltpu.CompilerParams(use_tc_tiling_on_sc=False,
                                           needs_layout_passes=False),
      scratch_types=[
          pltpu.VMEM((steps, IDXW), jnp.int32),
          pltpu.VMEM((32, 2048), jnp.float32),
          pltpu.VMEM((IDXW, 32), jnp.float32),
      ],
  )
  def k(w_hbm, bi_hbm, out_hbm, bi_v, reg, sel_v):
    w = lax.axis_index("s") * NC + lax.axis_index("c")
    pltpu.sync_copy(bi_hbm.at[w], bi_v)
    pltpu.sync_copy(w_hbm.at[pl.ds(0, s2), pl.ds(0, RD)],
                    reg.at[pl.ds(0, s2), pl.ds(0, RD)])

    def step(s, carry):
      def jc(q, c2):
        rvec = lax.iota(jnp.int32, 16) + q * 16
        bivec = bi_v[s, pl.ds(q * 16, 16)]

        def tcol(t, c3):
          tv = jnp.full((16,), 0, jnp.int32) + t
          vals = plsc.load_gather(reg, [tv, bivec])
          plsc.store_scatter(sel_v, [rvec, tv], vals)
          return c3

        lax.fori_loop(0, s2, tcol, 0)
        return c2

      lax.fori_loop(0, IDXW // 16, jc, 0)
      pltpu.sync_copy(sel_v.at[pl.ds(0, IDXW), pl.ds(0, s2)],
                      out_hbm.at[pl.ds((w * steps + s) * IDXW, IDXW)])
      return carry

    lax.fori_loop(0, steps, step, 0)

  return k(wideT, bi3)


def _sc_nash(g2, wideT, mi3, steps, s2, blk):
  @functools.partial(
      pl.kernel,
      out_type=jax.ShapeDtypeStruct((NW, 16), jnp.float32),
      mesh=_sc_mesh(),
      compiler_params=pltpu.CompilerParams(use_tc_tiling_on_sc=False,
                                           needs_layout_passes=False),
      scratch_types=[
          pltpu.VMEM((steps, IDXW), jnp.int32),
          pltpu.VMEM((IDXW, 32), jnp.float32),
          pltpu.VMEM((IDXW, 32), jnp.float32),
          pltpu.VMEM((32, IDXW), jnp.float32),
          pltpu.VMEM((32, IDXW), jnp.float32),
          pltpu.VMEM((16,), jnp.float32),
          pltpu.SemaphoreType.DMA,
          pltpu.SemaphoreType.DMA,
          pltpu.SemaphoreType.DMA,
          pltpu.SemaphoreType.DMA,
      ],
  )
  def k(g2_hbm, w_hbm, mi_hbm, out_hbm, mi_v, gb0, gb1, mb0, mb1, acc,
        sg0, sg1, sm0, sm1):
    w = lax.axis_index("s") * NC + lax.axis_index("c")
    pltpu.sync_copy(mi_hbm.at[w], mi_v)
    acc[...] = jnp.zeros((16,), jnp.float32)

    def wslice(s):
      p0 = (w * steps + s) * IDXW
      bk = p0 // blk
      return w_hbm.at[pl.ds(bk * s2, s2), pl.ds(p0 - bk * blk, IDXW)]

    def fire(s, gb, mb, sg, sm):
      pltpu.async_copy(g2_hbm.at[mi_v.at[s]], gb, sg)
      pltpu.async_copy(wslice(s), mb, sm)

    def wait(s, gb, mb, sg, sm):
      pltpu.make_async_copy(g2_hbm.at[mi_v.at[s]], gb, sg).wait()
      pltpu.make_async_copy(wslice(s), mb, sm).wait()

    def compute(gb, mb):
      def ft(j, c2):
        jv = jnp.full((16,), 0, jnp.int32) + j
        t = acc[...]
        for q in range(IDXW // 16):
          a = mb[j, pl.ds(q * 16, 16)]
          rvec = lax.iota(jnp.int32, 16) + q * 16
          b2 = plsc.load_gather(gb, [rvec, jv])
          dd = a - b2
          t = t + dd * dd
        acc[...] = t
        return c2

      lax.fori_loop(0, s2, ft, 0, unroll=4)

    fire(0, gb0, mb0, sg0, sm0)

    def body(t, carry):
      sa = 2 * t
      sb = sa + 1
      fire(sb, gb1, mb1, sg1, sm1)
      wait(sa, gb0, mb0, sg0, sm0)
      compute(gb0, mb0)

      @pl.when(sa + 2 < steps)
      def _():
        fire(sa + 2, gb0, mb0, sg0, sm0)

      wait(sb, gb1, mb1, sg1, sm1)
      compute(gb1, mb1)
      return carry

    lax.fori_loop(0, steps // 2, body, 0)
    pltpu.sync_copy(acc, out_hbm.at[w])

  return k(g2, wideT, mi3)


def _tc_mlp(medde, lab, Wa, ba, Wb, bb, Wc, bc, b_real):
  BP = medde.shape[0]
  BLK = 8192
  G = BP // BLK
  s2 = Wc.shape[0]
  s_dim = s2 // 2

  def body(x_ref, lab_ref, Wa_r, ba_r, Wb_r, bb_r, Wc_r, bc_r,
           pay_ref, msds_ref, bce_ref):
    i = pl.program_id(0)

    def dnT(w_ref, xT, dims):
      return lax.dot_general(w_ref[...], xT, (dims, ((), ())),
                             preferred_element_type=jnp.float32)

    pmdT = dnT(Wa_r, x_ref[...], ((1,), (1,))) + ba_r[...]
    hT = jnp.maximum(dnT(Wb_r, pmdT, ((1,), (0,))) + bb_r[...], 0.0)
    msdsT = dnT(Wc_r, hT, ((1,), (0,))) + bc_r[...]
    msT = msdsT[:s_dim, :]
    dsT = msdsT[s_dim:, :]
    num = jnp.sum(msT * dsT, axis=0, keepdims=True)
    den = jnp.sqrt(jnp.sum(msT * msT, axis=0, keepdims=True)) * \
        jnp.sqrt(jnp.sum(dsT * dsT, axis=0, keepdims=True))
    x = num / den
    pay_ref[...] = x[None]
    msds_ref[...] = msdsT
    y = lab_ref[0]
    bce = jnp.maximum(x, 0.0) - x * y + jnp.log1p(jnp.exp(-jnp.abs(x)))
    cidx = i * BLK + lax.broadcasted_iota(jnp.int32, (1, BLK), 1)
    s = jnp.sum(jnp.where(cidx < b_real, bce, 0.0), axis=(0, 1),
                keepdims=True)
    bce_ref[...] = jnp.where(i == 0, s, bce_ref[...] + s)

  rep = lambda shape: pl.BlockSpec(shape, lambda i: tuple(0 for _ in shape))
  return pl.pallas_call(
      body,
      grid=(G,),
      in_specs=[
          pl.BlockSpec((BLK, medde.shape[1]), lambda i: (i, 0)),
          pl.BlockSpec((1, 1, BLK), lambda i: (i, 0, 0)),
          rep(Wa.shape), rep(ba.shape), rep(Wb.shape), rep(bb.shape),
          rep(Wc.shape), rep(bc.shape),
      ],
      out_specs=[
          pl.BlockSpec((1, 1, BLK), lambda i: (i, 0, 0)),
          pl.BlockSpec((s2, BLK), lambda i: (i, 0)),
          pl.BlockSpec((1, 1), lambda i: (0, 0)),
      ],
      out_shape=[
          jax.ShapeDtypeStruct((G, 1, BLK), jnp.float32),
          jax.ShapeDtypeStruct((G * s2, BLK), jnp.float32),
          jax.ShapeDtypeStruct((1, 1), jnp.float32),
      ],
  )(medde, lab, Wa, ba, Wb, bb, Wc, bc)


def _tc_unpack(acc, T):
  def body(a_ref, o_ref):
    mx = jnp.max(a_ref[...], axis=0, keepdims=True)
    o_ref[...] = jnp.int32(2047) - (mx & jnp.int32(2047))

  return pl.pallas_call(
      body,
      out_shape=jax.ShapeDtypeStruct((1, T), jnp.int32),
  )(acc)


def _tc_loss(partials, padblk, g2row, bce, b_real, s_dim):
  def body(p_ref, pb_ref, gr_ref, bce_ref, out_ref):
    tot = jnp.sum(p_ref[...], axis=(0, 1), keepdims=True)
    dd = pb_ref[...] - gr_ref[...]
    pad_sum = jnp.sum(dd * dd, axis=(0, 1), keepdims=True)
    out_ref[...] = (tot - pad_sum) / (2.0 * b_real * s_dim) + \
        bce_ref[...] / b_real

  return pl.pallas_call(
      body,
      out_shape=jax.ShapeDtypeStruct((1, 1), jnp.float32),
  )(partials, padblk, g2row, bce)


def kernel(miRNA_embeddings, disease_embeddings, miRNA_index, disease_index,
           true_labels, W_m, b_m, W_d, b_d, W_ms1, b_ms1, W_ms2, b_ms2,
           W_ds1, b_ds1, W_ds2, b_ds2):
  num_m = miRNA_embeddings.shape[0]
  num_d = disease_embeddings.shape[0]
  B = miRNA_index.shape[0]
  s_dim = W_ms2.shape[0]

  BP = 16384 * (-(-B // 16384))
  steps = BP // (NW * IDXW)
  pad = BP - B
  mi = miRNA_index.astype(jnp.int32)
  di = disease_index.astype(jnp.int32)
  mi_g = jnp.pad(mi, (0, pad))
  di_g = jnp.pad(di, (0, pad))
  mi_s = jnp.pad(mi, (0, pad), constant_values=num_m)
  lab = jnp.pad(true_labels, (0, pad)).reshape(BP // 8192, 1, 8192)

  mi2 = mi_g.reshape(NW, steps, IDXW)
  di2 = di_g.reshape(NW, steps, IDXW)

  medde = _sc_gather2(miRNA_embeddings, disease_embeddings, mi2, di2, steps)

  h = W_m.shape[0]
  fm = W_m.shape[1]
  fd = W_d.shape[1]
  s_dim = W_ms2.shape[0]
  Wa = jnp.zeros((2 * h, fm + fd), jnp.float32)
  Wa = Wa.at[:h, :fm].set(W_m).at[h:, fm:].set(W_d)
  ba = jnp.concatenate([b_m, b_d]).reshape(2 * h, 1)
  Wb = jnp.zeros((2 * h, 2 * h), jnp.float32)
  Wb = Wb.at[:h, :h].set(W_ms1).at[h:, h:].set(W_ds1)
  bb = jnp.concatenate([b_ms1, b_ds1]).reshape(2 * h, 1)
  Wc = jnp.zeros((2 * s_dim, 2 * h), jnp.float32)
  Wc = Wc.at[:s_dim, :h].set(W_ms2).at[s_dim:, h:].set(W_ds2)
  bc = jnp.concatenate([b_ms2, b_ds2]).reshape(2 * s_dim, 1)

  pay, msds, bce = _tc_mlp(medde, lab, Wa, ba, Wb, bb, Wc, bc, B)

  bi_n = NW * IDXW * (-(-num_m // (NW * IDXW)))
  bi_steps = bi_n // (NW * IDXW)
  acc = _sc_segmax(pay.reshape(NW, steps, IDXW),
                   mi_s.reshape(NW, steps, IDXW), di2, steps, bi_n)
  bi2 = _tc_unpack(acc, bi_n)

  s2 = 2 * s_dim
  bi_pad = bi2.reshape(NW, bi_steps, IDXW)
  g2 = _sc_gsel(msds, bi_pad, bi_steps, s2, num_d)
  partials = _sc_nash(g2, msds, mi2, steps, s2, 8192)

  bk = B // 8192
  padT = msds[bk * s2:(bk + 1) * s2, B - bk * 8192:]
  g2col = jnp.transpose(g2[0:1, :])
  loss = _tc_loss(partials, padT, g2col, bce, B, s_dim)
  return (pay.reshape(-1)[:B], loss[0, 0])


# speedup vs baseline: 10.5856x; 1.0041x over previous
import functools

import jax
import jax.numpy as jnp
from jax import lax
from jax.experimental import pallas as pl
from jax.experimental.pallas import tpu as pltpu
from jax.experimental.pallas import tpu_sc as plsc

NC, NS = 2, 16
NW = NC * NS
IDXW = 128


def _sc_mesh():
  return plsc.VectorSubcoreMesh(
      core_axis_name="c", subcore_axis_name="s",
      num_cores=NC, num_subcores=NS)


def _sc_gather(table, idx3, steps):
  _, D = table.shape
  N = NW * steps * IDXW

  @functools.partial(
      pl.kernel,
      out_type=jax.ShapeDtypeStruct((N, D), jnp.float32),
      mesh=_sc_mesh(),
      compiler_params=pltpu.CompilerParams(use_tc_tiling_on_sc=False),
      scratch_types=[
          pltpu.VMEM((steps, IDXW), jnp.int32),
          pltpu.VMEM((IDXW, D), jnp.float32),
          pltpu.SemaphoreType.DMA,
      ],
  )
  def k(table_hbm, idx_hbm, out_hbm, idx_v, rows_v, sem):
    w = lax.axis_index("s") * NC + lax.axis_index("c")
    pltpu.sync_copy(idx_hbm.at[w], idx_v)

    def body(s, carry):
      pltpu.async_copy(table_hbm.at[idx_v.at[s]], rows_v, sem).wait()
      pltpu.sync_copy(rows_v, out_hbm.at[pl.ds((w * steps + s) * IDXW, IDXW)])
      return carry

    lax.fori_loop(0, steps, body, 0)

  return k(table, idx3)


def _sc_gatherp(table, idx3, steps):
  _, D = table.shape
  N = NW * steps * IDXW

  @functools.partial(
      pl.kernel,
      out_type=jax.ShapeDtypeStruct((N, D), jnp.float32),
      mesh=_sc_mesh(),
      compiler_params=pltpu.CompilerParams(use_tc_tiling_on_sc=False),
      scratch_types=[
          pltpu.VMEM((steps, IDXW), jnp.int32),
          pltpu.VMEM((IDXW, D), jnp.float32),
          pltpu.VMEM((IDXW, D), jnp.float32),
          pltpu.SemaphoreType.DMA,
          pltpu.SemaphoreType.DMA,
      ],
  )
  def k(t_hbm, idx_hbm, out_hbm, idx_v, r0, r1, s0, s1):
    w = lax.axis_index("s") * NC + lax.axis_index("c")
    base = w * steps
    pltpu.sync_copy(idx_hbm.at[w], idx_v)
    pltpu.async_copy(t_hbm.at[idx_v.at[0]], r0, s0)

    def body(t, carry):
      sa = 2 * t
      sb = 2 * t + 1
      pltpu.async_copy(t_hbm.at[idx_v.at[sb]], r1, s1)
      pltpu.make_async_copy(t_hbm.at[idx_v.at[sa]], r0, s0).wait()
      pltpu.sync_copy(r0, out_hbm.at[pl.ds((base + sa) * IDXW, IDXW)])

      @pl.when(sa + 2 < steps)
      def _():
        pltpu.async_copy(t_hbm.at[idx_v.at[sa + 2]], r0, s0)

      pltpu.make_async_copy(t_hbm.at[idx_v.at[sb]], r1, s1).wait()
      pltpu.sync_copy(r1, out_hbm.at[pl.ds((base + sb) * IDXW, IDXW)])
      return carry

    lax.fori_loop(0, steps // 2, body, 0)

  return k(table, idx3)


def _sc_gather2(tab_a, tab_b, idx_a3, idx_b3, steps):
  Da = tab_a.shape[1]
  Db = tab_b.shape[1]
  N = NW * steps * IDXW
  SS = 2 * IDXW

  @functools.partial(
      pl.kernel,
      out_type=jax.ShapeDtypeStruct((N, Da + Db), jnp.float32),
      mesh=_sc_mesh(),
      compiler_params=pltpu.CompilerParams(use_tc_tiling_on_sc=False),
      scratch_types=[
          pltpu.VMEM((steps, IDXW), jnp.int32),
          pltpu.VMEM((steps, IDXW), jnp.int32),
          pltpu.VMEM((SS, Da), jnp.float32),
          pltpu.VMEM((SS, Da), jnp.float32),
          pltpu.VMEM((SS, Db), jnp.float32),
          pltpu.VMEM((SS, Db), jnp.float32),
          pltpu.SemaphoreType.DMA,
          pltpu.SemaphoreType.DMA,
          pltpu.SemaphoreType.DMA,
          pltpu.SemaphoreType.DMA,
      ],
  )
  def k(ta_hbm, tb_hbm, ia_hbm, ib_hbm, out_hbm, ia_v, ib_v, ra0, ra1,
        rb0, rb1, sa0, sa1, sb0, sb1):
    w = lax.axis_index("s") * NC + lax.axis_index("c")
    base = w * steps
    US = steps // 2
    pltpu.sync_copy(ia_hbm.at[w], ia_v)
    pltpu.sync_copy(ib_hbm.at[w], ib_v)

    def fire(u, ra, rb, sa, sb):
      for h in range(2):
        pltpu.async_copy(ta_hbm.at[ia_v.at[2 * u + h]],
                         ra.at[pl.ds(h * IDXW, IDXW)], sa)
        pltpu.async_copy(tb_hbm.at[ib_v.at[2 * u + h]],
                         rb.at[pl.ds(h * IDXW, IDXW)], sb)

    def wait_write(u, ra, rb, sa, sb):
      for h in range(2):
        pltpu.make_async_copy(ta_hbm.at[ia_v.at[2 * u + h]],
                              ra.at[pl.ds(h * IDXW, IDXW)], sa).wait()
        pltpu.make_async_copy(tb_hbm.at[ib_v.at[2 * u + h]],
                              rb.at[pl.ds(h * IDXW, IDXW)], sb).wait()
      r0 = (base + 2 * u) * IDXW
      pltpu.sync_copy(ra, out_hbm.at[pl.ds(r0, SS), pl.ds(0, Da)])
      pltpu.sync_copy(rb, out_hbm.at[pl.ds(r0, SS), pl.ds(Da, Db)])

    fire(0, ra0, rb0, sa0, sb0)

    def body(g, carry):
      u0 = 2 * g
      u1 = u0 + 1
      fire(u1, ra1, rb1, sa1, sb1)
      wait_write(u0, ra0, rb0, sa0, sb0)

      @pl.when(u0 + 2 < US)
      def _():
        fire(u0 + 2, ra0, rb0, sa0, sb0)

      wait_write(u1, ra1, rb1, sa1, sb1)
      return carry

    lax.fori_loop(0, US // 2, body, 0)

  return k(tab_a, tab_b, idx_a3, idx_b3)


def _sc_segmax(vals3, mi3, di3, steps, T):
  seg = T // NS

  @functools.partial(
      pl.kernel,
      out_type=jax.ShapeDtypeStruct((NC, T), jnp.int32),
      mesh=_sc_mesh(),
      compiler_params=pltpu.CompilerParams(use_tc_tiling_on_sc=False,
                                           needs_layout_passes=False),
      scratch_types=[
          pltpu.VMEM((steps, IDXW), jnp.float32),
          pltpu.VMEM((steps, IDXW), jnp.int32),
          pltpu.VMEM((steps, IDXW), jnp.int32),
          pltpu.VMEM((T,), jnp.int32),
          pltpu.VMEM((seg,), jnp.int32),
          pltpu.VMEM((seg,), jnp.int32),
          pltpu.VMEM_SHARED((NS, T), jnp.int32),
      ],
  )
  def k(val_hbm, mi_hbm, di_hbm, out_hbm, val_v, mi_v, di_v, tbl, mseg,
        tseg, spm):
    c = lax.axis_index("c")
    sid = lax.axis_index("s")
    w = sid * NC + c

    def init(i, carry):
      tbl[pl.ds(i * 16, 16)] = jnp.full((16,), 2047, jnp.int32)
      return carry

    lax.fori_loop(0, T // 16, init, 0)
    pltpu.sync_copy(val_hbm.at[w], val_v)
    pltpu.sync_copy(mi_hbm.at[w], mi_v)
    pltpu.sync_copy(di_hbm.at[w], di_v)

    def row(s, carry):
      def vec(kk, carry2):
        m = mi_v[s, pl.ds(kk * 16, 16)]
        d = di_v[s, pl.ds(kk * 16, 16)]
        v = val_v[s, pl.ds(kk * 16, 16)]
        b = plsc.bitcast(v, jnp.int32)
        key32 = jnp.where(b >= 0, b, b ^ jnp.int32(0x7FFFFFFF))
        key = (key32 & jnp.int32(-2048)) | (jnp.int32(2047) - d)

        def w_cond(mask):
          return jnp.max(jnp.where(mask, 1, 0)) > 0

        def w_body(mask):
          plsc.store_scatter(tbl, [m], key, mask=mask)
          cur = plsc.load_gather(tbl, [m])
          return mask & (cur < key)

        lax.while_loop(w_cond, w_body,
                       key > plsc.load_gather(tbl, [m]))
        return carry2

      lax.fori_loop(0, IDXW // 16, vec, carry)
      return carry

    lax.fori_loop(0, steps, row, 0)

    pltpu.sync_copy(tbl, spm.at[sid])
    plsc.subcore_barrier()
    pltpu.sync_copy(spm.at[0, pl.ds(sid * seg, seg)], mseg)

    def merge(j, carry):
      pltpu.sync_copy(spm.at[j, pl.ds(sid * seg, seg)], tseg)

      def mvec(i, carry2):
        mseg[pl.ds(i * 16, 16)] = jnp.maximum(
            mseg[pl.ds(i * 16, 16)], tseg[pl.ds(i * 16, 16)])
        return carry2

      lax.fori_loop(0, seg // 16, mvec, carry)
      return carry

    lax.fori_loop(1, NS, merge, 0)
    pltpu.sync_copy(mseg, out_hbm.at[c, pl.ds(sid * seg, seg)])

  return k(vals3, mi3, di3)


def _sc_gsel(wideT, bi3, steps, s2, num_d):
  N = NW * steps * IDXW
  RD = -(-num_d // IDXW) * IDXW

  @functools.partial(
      pl.kernel,
      out_type=jax.ShapeDtypeStruct((N, s2), jnp.float32),
      mesh=_sc_mesh(),
      compiler_params=pltpu.CompilerParams(use_tc_tiling_on_sc=False,
                                           needs_layout_passes=False),
      scratch_types=[
          pltpu.VMEM((steps, IDXW), jnp.int32),
          pltpu.VMEM((32, 2048), jnp.float32),
          pltpu.VMEM((IDXW, 32), jnp.float32),
      ],
  )
  def k(w_hbm, bi_hbm, out_hbm, bi_v, reg, sel_v):
    w = lax.axis_index("s") * NC + lax.axis_index("c")
    pltpu.sync_copy(bi_hbm.at[w], bi_v)
    pltpu.sync_copy(w_hbm.at[pl.ds(0, s2), pl.ds(0, RD)],
                    reg.at[pl.ds(0, s2), pl.ds(0, RD)])

    def step(s, carry):
      def jc(q, c2):
        rvec = lax.iota(jnp.int32, 16) + q * 16
        bivec = bi_v[s, pl.ds(q * 16, 16)]

        def tcol(t, c3):
          tv = jnp.full((16,), 0, jnp.int32) + t
          vals = plsc.load_gather(reg, [tv, bivec])
          plsc.store_scatter(sel_v, [rvec, tv], vals)
          return c3

        lax.fori_loop(0, s2, tcol, 0)
        return c2

      lax.fori_loop(0, IDXW // 16, jc, 0)
      pltpu.sync_copy(sel_v.at[pl.ds(0, IDXW), pl.ds(0, s2)],
                      out_hbm.at[pl.ds((w * steps + s) * IDXW, IDXW)])
      return carry

    lax.fori_loop(0, steps, step, 0)

  return k(wideT, bi3)


def _sc_nash(g2, wideT, mi3, steps, s2, blk):
  @functools.partial(
      pl.kernel,
      out_type=jax.ShapeDtypeStruct((NW, 16), jnp.float32),
      mesh=_sc_mesh(),
      compiler_params=pltpu.CompilerParams(use_tc_tiling_on_sc=False,
                                           needs_layout_passes=False),
      scratch_types=[
          pltpu.VMEM((steps, IDXW), jnp.int32),
          pltpu.VMEM((IDXW, 32), jnp.float32),
          pltpu.VMEM((IDXW, 32), jnp.float32),
          pltpu.VMEM((32, IDXW), jnp.float32),
          pltpu.VMEM((32, IDXW), jnp.float32),
          pltpu.VMEM((16,), jnp.float32),
          pltpu.SemaphoreType.DMA,
          pltpu.SemaphoreType.DMA,
          pltpu.SemaphoreType.DMA,
          pltpu.SemaphoreType.DMA,
      ],
  )
  def k(g2_hbm, w_hbm, mi_hbm, out_hbm, mi_v, gb0, gb1, mb0, mb1, acc,
        sg0, sg1, sm0, sm1):
    w = lax.axis_index("s") * NC + lax.axis_index("c")
    pltpu.sync_copy(mi_hbm.at[w], mi_v)
    acc[...] = jnp.zeros((16,), jnp.float32)

    def wslice(s):
      p0 = (w * steps + s) * IDXW
      bk = p0 // blk
      return w_hbm.at[pl.ds(bk * s2, s2), pl.ds(p0 - bk * blk, IDXW)]

    def fire(s, gb, mb, sg, sm):
      pltpu.async_copy(g2_hbm.at[mi_v.at[s]], gb, sg)
      pltpu.async_copy(wslice(s), mb, sm)

    def wait(s, gb, mb, sg, sm):
      pltpu.make_async_copy(g2_hbm.at[mi_v.at[s]], gb, sg).wait()
      pltpu.make_async_copy(wslice(s), mb, sm).wait()

    def compute(gb, mb):
      def ft(j, c2):
        jv = jnp.full((16,), 0, jnp.int32) + j
        t = acc[...]
        for q in range(IDXW // 16):
          a = mb[j, pl.ds(q * 16, 16)]
          rvec = lax.iota(jnp.int32, 16) + q * 16
          b2 = plsc.load_gather(gb, [rvec, jv])
          dd = a - b2
          t = t + dd * dd
        acc[...] = t
        return c2

      lax.fori_loop(0, s2, ft, 0, unroll=4)

    fire(0, gb0, mb0, sg0, sm0)

    def body(t, carry):
      sa = 2 * t
      sb = sa + 1
      fire(sb, gb1, mb1, sg1, sm1)
      wait(sa, gb0, mb0, sg0, sm0)
      compute(gb0, mb0)

      @pl.when(sa + 2 < steps)
      def _():
        fire(sa + 2, gb0, mb0, sg0, sm0)

      wait(sb, gb1, mb1, sg1, sm1)
      compute(gb1, mb1)
      return carry

    lax.fori_loop(0, steps // 2, body, 0)
    pltpu.sync_copy(acc, out_hbm.at[w])

  return k(g2, wideT, mi3)


def _tc_mlp(medde, lab, Wa, ba, Wb, bb, Wc, bc, b_real):
  BP = medde.shape[0]
  BLK = 8192
  G = BP // BLK
  s2 = Wc.shape[0]
  s_dim = s2 // 2

  def body(x_ref, lab_ref, Wa_r, ba_r, Wb_r, bb_r, Wc_r, bc_r,
           pay_ref, msds_ref, bce_ref):
    i = pl.program_id(0)

    def dnT(w_ref, xT, dims):
      return lax.dot_general(w_ref[...], xT, (dims, ((), ())),
                             preferred_element_type=jnp.float32)

    pmdT = dnT(Wa_r, x_ref[...], ((1,), (1,))) + ba_r[...]
    hT = jnp.maximum(dnT(Wb_r, pmdT, ((1,), (0,))) + bb_r[...], 0.0)
    msdsT = dnT(Wc_r, hT, ((1,), (0,))) + bc_r[...]
    msT = msdsT[:s_dim, :]
    dsT = msdsT[s_dim:, :]
    num = jnp.sum(msT * dsT, axis=0, keepdims=True)
    den = jnp.sqrt(jnp.sum(msT * msT, axis=0, keepdims=True)) * \
        jnp.sqrt(jnp.sum(dsT * dsT, axis=0, keepdims=True))
    x = num / den
    pay_ref[...] = x[None]
    msds_ref[...] = msdsT
    y = lab_ref[0]
    bce = jnp.maximum(x, 0.0) - x * y + jnp.log1p(jnp.exp(-jnp.abs(x)))
    cidx = i * BLK + lax.broadcasted_iota(jnp.int32, (1, BLK), 1)
    s = jnp.sum(jnp.where(cidx < b_real, bce, 0.0), axis=(0, 1),
                keepdims=True)
    bce_ref[...] = jnp.where(i == 0, s, bce_ref[...] + s)

  rep = lambda shape: pl.BlockSpec(shape, lambda i: tuple(0 for _ in shape))
  return pl.pallas_call(
      body,
      grid=(G,),
      in_specs=[
          pl.BlockSpec((BLK, medde.shape[1]), lambda i: (i, 0)),
          pl.BlockSpec((1, 1, BLK), lambda i: (i, 0, 0)),
          rep(Wa.shape), rep(ba.shape), rep(Wb.shape), rep(bb.shape),
          rep(Wc.shape), rep(bc.shape),
      ],
      out_specs=[
          pl.BlockSpec((1, 1, BLK), lambda i: (i, 0, 0)),
          pl.BlockSpec((s2, BLK), lambda i: (i, 0)),
          pl.BlockSpec((1, 1), lambda i: (0, 0)),
      ],
      out_shape=[
          jax.ShapeDtypeStruct((G, 1, BLK), jnp.float32),
          jax.ShapeDtypeStruct((G * s2, BLK), jnp.float32),
          jax.ShapeDtypeStruct((1, 1), jnp.float32),
      ],
  )(medde, lab, Wa, ba, Wb, bb, Wc, bc)


def _tc_unpack(acc, T):
  def body(a_ref, o_ref):
    mx = jnp.max(a_ref[...], axis=0, keepdims=True)
    o_ref[...] = jnp.int32(2047) - (mx & jnp.int32(2047))

  return pl.pallas_call(
      body,
      out_shape=jax.ShapeDtypeStruct((1, T), jnp.int32),
  )(acc)


def _tc_loss(partials, padblk, g2row, bce, b_real, s_dim):
  def body(p_ref, pb_ref, gr_ref, bce_ref, out_ref):
    tot = jnp.sum(p_ref[...], axis=(0, 1), keepdims=True)
    dd = pb_ref[...] - gr_ref[...]
    pad_sum = jnp.sum(dd * dd, axis=(0, 1), keepdims=True)
    out_ref[...] = (tot - pad_sum) / (2.0 * b_real * s_dim) + \
        bce_ref[...] / b_real

  return pl.pallas_call(
      body,
      out_shape=jax.ShapeDtypeStruct((1, 1), jnp.float32),
  )(partials, padblk, g2row, bce)


def kernel(miRNA_embeddings, disease_embeddings, miRNA_index, disease_index,
           true_labels, W_m, b_m, W_d, b_d, W_ms1, b_ms1, W_ms2, b_ms2,
           W_ds1, b_ds1, W_ds2, b_ds2):
  num_m = miRNA_embeddings.shape[0]
  num_d = disease_embeddings.shape[0]
  B = miRNA_index.shape[0]
  s_dim = W_ms2.shape[0]

  BP = 16384 * (-(-B // 16384))
  steps = BP // (NW * IDXW)
  pad = BP - B
  mi = miRNA_index.astype(jnp.int32)
  di = disease_index.astype(jnp.int32)
  mi_g = jnp.pad(mi, (0, pad))
  di_g = jnp.pad(di, (0, pad))
  mi_s = jnp.pad(mi, (0, pad), constant_values=num_m)
  lab = jnp.pad(true_labels, (0, pad)).reshape(BP // 8192, 1, 8192)

  mi2 = mi_g.reshape(NW, steps, IDXW)
  di2 = di_g.reshape(NW, steps, IDXW)

  medde = _sc_gather2(miRNA_embeddings, disease_embeddings, mi2, di2, steps)

  h = W_m.shape[0]
  fm = W_m.shape[1]
  fd = W_d.shape[1]
  s_dim = W_ms2.shape[0]
  Wa = jnp.zeros((2 * h, fm + fd), jnp.float32)
  Wa = Wa.at[:h, :fm].set(W_m).at[h:, fm:].set(W_d)
  ba = jnp.concatenate([b_m, b_d]).reshape(2 * h, 1)
  Wb = jnp.zeros((2 * h, 2 * h), jnp.float32)
  Wb = Wb.at[:h, :h].set(W_ms1).at[h:, h:].set(W_ds1)
  bb = jnp.concatenate([b_ms1, b_ds1]).reshape(2 * h, 1)
  Wc = jnp.zeros((2 * s_dim, 2 * h), jnp.float32)
  Wc = Wc.at[:s_dim, :h].set(W_ms2).at[s_dim:, h:].set(W_ds2)
  bc = jnp.concatenate([b_ms2, b_ds2]).reshape(2 * s_dim, 1)

  pay, msds, bce = _tc_mlp(medde, lab, Wa, ba, Wb, bb, Wc, bc, B)

  bi_n = NW * IDXW * (-(-num_m // (NW * IDXW)))
  bi_steps = bi_n // (NW * IDXW)
  acc = _sc_segmax(pay.reshape(NW, steps, IDXW),
                   mi_s.reshape(NW, steps, IDXW), di2, steps, bi_n)
  bi2 = _tc_unpack(acc, bi_n)

  s2 = 2 * s_dim
  bi_pad = bi2.reshape(NW, bi_steps, IDXW)
  g2 = _sc_gsel(msds, bi_pad, bi_steps, s2, num_d)
  partials = _sc_nash(g2, msds, mi2, steps, s2, 8192)

  bk = B // 8192
  padT = msds[bk * s2:(bk + 1) * s2, B - bk * 8192:]
  g2col = jnp.transpose(g2[0:1, :])
  loss = _tc_loss(partials, padT, g2col, bce, B, s_dim)
  return (pay.reshape(-1)[:B], loss[0, 0])
